# baseline (speedup 1.0000x reference)
import functools

import jax
import jax.numpy as jnp
from jax import lax
from jax.experimental import pallas as pl
from jax.experimental.pallas import tpu as pltpu
from jax.experimental.pallas import tpu_sc as plsc

N = 50000
N_OUT = 40000
E_EDGES = 800000
D_IN = 128
H_DIM = 64
HH = 32
C_CLS = 50

ROW_BLK = 1000
N_SUBC = 16
NPAD = 50048
TILE_ROWS = NPAD // N_SUBC
ZCHUNK = 184
EB = 80
EPAD = 819200
NBATCH = EPAD // EB // N_SUBC
G = 8
NG = NBATCH // G
NBUF = 4
CHS = 64
NCH = NBATCH // CHS
NGC = CHS // G

_sc_mesh = plsc.VectorSubcoreMesh(core_axis_name="c", subcore_axis_name="s")


@functools.partial(
    pl.kernel,
    mesh=_sc_mesh,
    compiler_params=pltpu.CompilerParams(use_tc_tiling_on_sc=False),
    out_type=jax.ShapeDtypeStruct((NPAD, 16), jnp.float32),
    scratch_types=[
        pltpu.VMEM((NBATCH, EB), jnp.int32),
        pltpu.VMEM((EB, 16), jnp.float32),
        pltpu.VMEM((ZCHUNK, 16), jnp.float32),
        pltpu.VMEM_SHARED((NPAD, 16), jnp.float32),
        pltpu.SemaphoreType.DMA,
        pltpu.SemaphoreType.DMA,
        pltpu.SemaphoreType.DMA,
        pltpu.SemaphoreType.DMA,
    ],
)
def _deg_sc(dst2, out_deg, dbuf, ones_v, zbuf, acc, sem0, sem1, sem2, sem3):
    c = lax.axis_index("c")
    s = lax.axis_index("s")
    sems = (sem0, sem1, sem2, sem3)

    @pl.when(c == 0)
    def _():
        def fill_ones(i, carry):
            ones_v[i, :] = jnp.ones((16,), jnp.float32)
            return carry
        lax.fori_loop(0, EB, fill_ones, 0)

        def fill_zero(i, carry):
            zbuf[i, :] = jnp.zeros((16,), jnp.float32)
            return carry
        lax.fori_loop(0, ZCHUNK, fill_zero, 0)

        row0 = s * TILE_ROWS
        def zero_acc(j, carry):
            pltpu.sync_copy(zbuf, acc.at[pl.ds(row0 + j * ZCHUNK, ZCHUNK)])
            return carry
        lax.fori_loop(0, TILE_ROWS // ZCHUNK, zero_acc, 0)
        pltpu.sync_copy(dst2.at[pl.ds(s * NBATCH, NBATCH)], dbuf)
        plsc.subcore_barrier()

        def ws_recon(b):
            pltpu.make_async_copy(ones_v, acc.at[dbuf.at[0]], sems[b]).wait()

        def group(gi, carry):
            base = gi * G
            sc = {}
            for j in range(G):
                b = j % NBUF
                if j < NBUF:
                    pl.when(gi > 0)(functools.partial(ws_recon, b))
                else:
                    sc[j - NBUF].wait()
                sc[j] = pltpu.async_copy(ones_v, acc.at[dbuf.at[base + j]],
                                         sems[b], add=True)
            return carry
        lax.fori_loop(0, NG, group, 0)
        for b in range(NBUF):
            ws_recon(b)
        plsc.subcore_barrier()

        pltpu.sync_copy(acc.at[pl.ds(row0, TILE_ROWS)],
                        out_deg.at[pl.ds(row0, TILE_ROWS)])


@functools.partial(
    pl.kernel,
    mesh=_sc_mesh,
    compiler_params=pltpu.CompilerParams(use_tc_tiling_on_sc=False),
    out_type=[jax.ShapeDtypeStruct((NPAD, HH), jnp.float32),
              jax.ShapeDtypeStruct((NPAD, HH), jnp.float32)],
    scratch_types=[
        pltpu.VMEM((CHS, EB), jnp.int32),
        pltpu.VMEM((CHS, EB), jnp.int32),
        pltpu.VMEM((EB, HH), jnp.float32),
        pltpu.VMEM((EB, HH), jnp.float32),
        pltpu.VMEM((EB, HH), jnp.float32),
        pltpu.VMEM((EB, HH), jnp.float32),
        pltpu.VMEM((ZCHUNK, HH), jnp.float32),
        pltpu.VMEM_SHARED((NPAD, HH), jnp.float32),
        pltpu.SemaphoreType.DMA,
        pltpu.SemaphoreType.DMA,
        pltpu.SemaphoreType.DMA,
        pltpu.SemaphoreType.DMA,
        pltpu.SemaphoreType.DMA,
        pltpu.SemaphoreType.DMA,
        pltpu.SemaphoreType.DMA,
        pltpu.SemaphoreType.DMA,
    ],
)
def _spmm_sc(ya, yb, src2, dst2, out_a, out_b, sbuf, dbuf,
             rows0, rows1, rows2, rows3, zbuf, acc,
             gsem0, gsem1, gsem2, gsem3, ssem0, ssem1, ssem2, ssem3):
    c = lax.axis_index("c")
    s = lax.axis_index("s")
    rows = (rows0, rows1, rows2, rows3)
    gsems = (gsem0, gsem1, gsem2, gsem3)
    ssems = (ssem0, ssem1, ssem2, ssem3)

    def fill_zero(i, carry):
        zbuf[i, pl.ds(0, 16)] = jnp.zeros((16,), jnp.float32)
        zbuf[i, pl.ds(16, 16)] = jnp.zeros((16,), jnp.float32)
        return carry
    lax.fori_loop(0, ZCHUNK, fill_zero, 0)

    row0 = s * TILE_ROWS
    def zero_acc(j, carry):
        pltpu.sync_copy(zbuf, acc.at[pl.ds(row0 + j * ZCHUNK, ZCHUNK)])
        return carry
    lax.fori_loop(0, TILE_ROWS // ZCHUNK, zero_acc, 0)
    plsc.subcore_barrier()

    def edge_pass(y_hbm):
        def ws_recon(b):
            pltpu.make_async_copy(rows[b], acc.at[dbuf.at[0]], ssems[b]).wait()

        def wg_recon(b):
            pltpu.make_async_copy(y_hbm.at[sbuf.at[0]], rows[b], gsems[b]).wait()

        def isc(t, b):
            return pltpu.async_copy(rows[b], acc.at[dbuf.at[t]], ssems[b],
                                    add=True)

        tile0 = s * NBATCH
        def chunk(ci, carry):
            pltpu.sync_copy(src2.at[pl.ds(tile0 + ci * CHS, CHS)], sbuf)
            pltpu.sync_copy(dst2.at[pl.ds(tile0 + ci * CHS, CHS)], dbuf)

            def group(gi, carry2):
                base = gi * G
                g, sc = {}, {}
                for j in range(G):
                    b = j % NBUF
                    if j < NBUF:
                        pl.when(gi > 0)(functools.partial(ws_recon, b))
                    else:
                        sc[j - NBUF].wait()
                    g[j] = pltpu.async_copy(y_hbm.at[sbuf.at[base + j]],
                                            rows[b], gsems[b])
                    if j == 0:
                        def head():
                            wg_recon(NBUF - 1)
                            isc(base - 1, NBUF - 1)
                        pl.when(gi > 0)(head)
                    else:
                        g[j - 1].wait()
                        sc[j - 1] = isc(base + j - 1, (j - 1) % NBUF)
                return carry2
            lax.fori_loop(0, NGC, group, 0)
            wg_recon((CHS - 1) % NBUF)
            isc(CHS - 1, (CHS - 1) % NBUF)
            for b in range(NBUF):
                ws_recon(b)
            return carry
        lax.fori_loop(0, NCH, chunk, 0)

    pl.when(c == 0)(lambda: edge_pass(ya))
    pl.when(c == 1)(lambda: edge_pass(yb))
    plsc.subcore_barrier()

    pl.when(c == 0)(lambda: pltpu.sync_copy(acc.at[pl.ds(row0, TILE_ROWS)],
                                            out_a.at[pl.ds(row0, TILE_ROWS)]))
    pl.when(c == 1)(lambda: pltpu.sync_copy(acc.at[pl.ds(row0, TILE_ROWS)],
                                            out_b.at[pl.ds(row0, TILE_ROWS)]))


def _dense0_body(f_ref, wred_ref, bred_ref, w0_ref, b0_ref, ae_ref, ya_ref, yb_ref):
    ae = jnp.dot(f_ref[...], wred_ref[...],
                 preferred_element_type=jnp.float32) + bred_ref[...]
    y0 = jnp.dot(ae, w0_ref[...], preferred_element_type=jnp.float32) + b0_ref[...]
    ae_ref[...] = ae
    ya_ref[...] = y0[:, :HH]
    yb_ref[...] = y0[:, HH:]


def _dense0(features, W_red, b_red, W0, b0):
    return pl.pallas_call(
        _dense0_body,
        grid=(N // ROW_BLK,),
        in_specs=[
            pl.BlockSpec((ROW_BLK, D_IN), lambda i: (i, 0)),
            pl.BlockSpec((D_IN, H_DIM), lambda i: (0, 0)),
            pl.BlockSpec((1, H_DIM), lambda i: (0, 0)),
            pl.BlockSpec((H_DIM, H_DIM), lambda i: (0, 0)),
            pl.BlockSpec((1, H_DIM), lambda i: (0, 0)),
        ],
        out_specs=[
            pl.BlockSpec((ROW_BLK, H_DIM), lambda i: (i, 0)),
            pl.BlockSpec((ROW_BLK, HH), lambda i: (i, 0)),
            pl.BlockSpec((ROW_BLK, HH), lambda i: (i, 0)),
        ],
        out_shape=[
            jax.ShapeDtypeStruct((N, H_DIM), jnp.float32),
            jax.ShapeDtypeStruct((N, HH), jnp.float32),
            jax.ShapeDtypeStruct((N, HH), jnp.float32),
        ],
    )(features, W_red, b_red.reshape(1, -1), W0, b0.reshape(1, -1))


def _dense1_body(sa_ref, sb_ref, deg_ref, w1_ref, b1_ref, x1_ref, ya_ref, yb_ref):
    inv = 1.0 / jnp.maximum(deg_ref[:, 0:1], 1.0)
    x1 = jnp.concatenate([sa_ref[...] * inv, sb_ref[...] * inv], axis=1)
    y1 = jnp.dot(x1, w1_ref[...], preferred_element_type=jnp.float32) + b1_ref[...]
    x1_ref[...] = x1
    ya_ref[...] = y1[:, :HH]
    yb_ref[...] = y1[:, HH:]


def _dense1(s0a, s0b, deg16, W1, b1):
    return pl.pallas_call(
        _dense1_body,
        grid=(N // ROW_BLK,),
        in_specs=[
            pl.BlockSpec((ROW_BLK, HH), lambda i: (i, 0)),
            pl.BlockSpec((ROW_BLK, HH), lambda i: (i, 0)),
            pl.BlockSpec((ROW_BLK, 16), lambda i: (i, 0)),
            pl.BlockSpec((H_DIM, H_DIM), lambda i: (0, 0)),
            pl.BlockSpec((1, H_DIM), lambda i: (0, 0)),
        ],
        out_specs=[
            pl.BlockSpec((ROW_BLK, H_DIM), lambda i: (i, 0)),
            pl.BlockSpec((ROW_BLK, HH), lambda i: (i, 0)),
            pl.BlockSpec((ROW_BLK, HH), lambda i: (i, 0)),
        ],
        out_shape=[
            jax.ShapeDtypeStruct((N, H_DIM), jnp.float32),
            jax.ShapeDtypeStruct((N, HH), jnp.float32),
            jax.ShapeDtypeStruct((N, HH), jnp.float32),
        ],
    )(s0a, s0b, deg16, W1, b1.reshape(1, -1))


def _final_body(ae_ref, x1_ref, sa_ref, sb_ref, deg_ref, wc_ref, bc_ref, out_ref):
    inv = 1.0 / jnp.maximum(deg_ref[:, 0:1], 1.0)
    x2 = jnp.concatenate([sa_ref[...] * inv, sb_ref[...] * inv], axis=1)
    m = (ae_ref[...] + x1_ref[...] + x2) * (1.0 / 3.0)
    z = jnp.dot(m, wc_ref[...], preferred_element_type=jnp.float32) + bc_ref[...]
    zmax = jnp.max(z, axis=1, keepdims=True)
    lse = jnp.log(jnp.sum(jnp.exp(z - zmax), axis=1, keepdims=True)) + zmax
    out_ref[...] = z - lse


def _final(all_emb, x1, s1a, s1b, deg16, W_cls, b_cls):
    return pl.pallas_call(
        _final_body,
        grid=(N_OUT // ROW_BLK,),
        in_specs=[
            pl.BlockSpec((ROW_BLK, H_DIM), lambda i: (i, 0)),
            pl.BlockSpec((ROW_BLK, H_DIM), lambda i: (i, 0)),
            pl.BlockSpec((ROW_BLK, HH), lambda i: (i, 0)),
            pl.BlockSpec((ROW_BLK, HH), lambda i: (i, 0)),
            pl.BlockSpec((ROW_BLK, 16), lambda i: (i, 0)),
            pl.BlockSpec((H_DIM, C_CLS), lambda i: (0, 0)),
            pl.BlockSpec((1, C_CLS), lambda i: (0, 0)),
        ],
        out_specs=pl.BlockSpec((ROW_BLK, C_CLS), lambda i: (i, 0)),
        out_shape=jax.ShapeDtypeStruct((N_OUT, C_CLS), jnp.float32),
    )(all_emb, x1, s1a, s1b, deg16, W_cls, b_cls.reshape(1, -1))


def kernel(features, edge_index, W_red, b_red, conv_weight_0, conv_bias_0,
           conv_weight_1, conv_bias_1, W_cls, b_cls):
    npad_e = EPAD - E_EDGES
    pad_src = jnp.zeros((npad_e,), jnp.int32)
    pad_dst = N + (jnp.arange(npad_e, dtype=jnp.int32) % (NPAD - N))
    src2 = jnp.concatenate([edge_index[0], pad_src]).reshape(EPAD // EB, EB)
    dst2 = jnp.concatenate([edge_index[1], pad_dst]).reshape(EPAD // EB, EB)
    deg16 = _deg_sc(dst2)
    all_emb, y0a, y0b = _dense0(features, W_red, b_red, conv_weight_0, conv_bias_0)
    s0a, s0b = _spmm_sc(y0a, y0b, src2, dst2)
    x1, y1a, y1b = _dense1(s0a, s0b, deg16, conv_weight_1, conv_bias_1)
    s1a, s1b = _spmm_sc(y1a, y1b, src2, dst2)
    return _final(all_emb, x1, s1a, s1b, deg16, W_cls, b_cls)


# speedup vs baseline: 1.0046x; 1.0046x over previous
import functools

import jax
import jax.numpy as jnp
from jax import lax
from jax.experimental import pallas as pl
from jax.experimental.pallas import tpu as pltpu
from jax.experimental.pallas import tpu_sc as plsc

N = 50000
N_OUT = 40000
E_EDGES = 800000
D_IN = 128
H_DIM = 64
HH = 32
C_CLS = 50

ROW_BLK = 1000
N_SUBC = 16
NPAD = 50048
TILE_ROWS = NPAD // N_SUBC
ZCHUNK = 184
EB = 80
EPAD = 819200
NBATCH = EPAD // EB // N_SUBC
G = 8
NG = NBATCH // G
NBUF = 4
SEB = 128
SNBATCH = EPAD // SEB // N_SUBC
SCH = 25
SNCH = SNBATCH // SCH

_sc_mesh = plsc.VectorSubcoreMesh(core_axis_name="c", subcore_axis_name="s")


@functools.partial(
    pl.kernel,
    mesh=_sc_mesh,
    compiler_params=pltpu.CompilerParams(use_tc_tiling_on_sc=False),
    out_type=jax.ShapeDtypeStruct((NPAD, 16), jnp.float32),
    scratch_types=[
        pltpu.VMEM((NBATCH, EB), jnp.int32),
        pltpu.VMEM((EB, 16), jnp.float32),
        pltpu.VMEM((ZCHUNK, 16), jnp.float32),
        pltpu.VMEM_SHARED((NPAD, 16), jnp.float32),
        pltpu.SemaphoreType.DMA,
        pltpu.SemaphoreType.DMA,
        pltpu.SemaphoreType.DMA,
        pltpu.SemaphoreType.DMA,
    ],
)
def _deg_sc(dst2, out_deg, dbuf, ones_v, zbuf, acc, sem0, sem1, sem2, sem3):
    c = lax.axis_index("c")
    s = lax.axis_index("s")
    sems = (sem0, sem1, sem2, sem3)

    @pl.when(c == 0)
    def _():
        def fill_ones(i, carry):
            ones_v[i, :] = jnp.ones((16,), jnp.float32)
            return carry
        lax.fori_loop(0, EB, fill_ones, 0)

        def fill_zero(i, carry):
            zbuf[i, :] = jnp.zeros((16,), jnp.float32)
            return carry
        lax.fori_loop(0, ZCHUNK, fill_zero, 0)

        row0 = s * TILE_ROWS
        def zero_acc(j, carry):
            pltpu.sync_copy(zbuf, acc.at[pl.ds(row0 + j * ZCHUNK, ZCHUNK)])
            return carry
        lax.fori_loop(0, TILE_ROWS // ZCHUNK, zero_acc, 0)
        pltpu.sync_copy(dst2.at[pl.ds(s * NBATCH, NBATCH)], dbuf)
        plsc.subcore_barrier()

        def ws_recon(b):
            pltpu.make_async_copy(ones_v, acc.at[dbuf.at[0]], sems[b]).wait()

        def group(gi, carry):
            base = gi * G
            sc = {}
            for j in range(G):
                b = j % NBUF
                if j < NBUF:
                    pl.when(gi > 0)(functools.partial(ws_recon, b))
                else:
                    sc[j - NBUF].wait()
                sc[j] = pltpu.async_copy(ones_v, acc.at[dbuf.at[base + j]],
                                         sems[b], add=True)
            return carry
        lax.fori_loop(0, NG, group, 0)
        for b in range(NBUF):
            ws_recon(b)
        plsc.subcore_barrier()

        pltpu.sync_copy(acc.at[pl.ds(row0, TILE_ROWS)],
                        out_deg.at[pl.ds(row0, TILE_ROWS)])


@functools.partial(
    pl.kernel,
    mesh=_sc_mesh,
    compiler_params=pltpu.CompilerParams(use_tc_tiling_on_sc=False),
    out_type=[jax.ShapeDtypeStruct((NPAD, HH), jnp.float32),
              jax.ShapeDtypeStruct((NPAD, HH), jnp.float32)],
    scratch_types=[
        pltpu.VMEM((SCH, SEB), jnp.int32),
        pltpu.VMEM((SCH, SEB), jnp.int32),
        pltpu.VMEM((SEB, HH), jnp.float32),
        pltpu.VMEM((SEB, HH), jnp.float32),
        pltpu.VMEM((ZCHUNK, HH), jnp.float32),
        pltpu.VMEM_SHARED((NPAD, HH), jnp.float32),
        pltpu.SemaphoreType.DMA,
        pltpu.SemaphoreType.DMA,
        pltpu.SemaphoreType.DMA,
        pltpu.SemaphoreType.DMA,
    ],
)
def _spmm_sc(ya, yb, src2, dst2, out_a, out_b, sbuf, dbuf,
             rows0, rows1, zbuf, acc, gsem0, gsem1, ssem0, ssem1):
    c = lax.axis_index("c")
    s = lax.axis_index("s")
    rows = (rows0, rows1)
    gsems = (gsem0, gsem1)
    ssems = (ssem0, ssem1)

    def fill_zero(i, carry):
        zbuf[i, pl.ds(0, 16)] = jnp.zeros((16,), jnp.float32)
        zbuf[i, pl.ds(16, 16)] = jnp.zeros((16,), jnp.float32)
        return carry
    lax.fori_loop(0, ZCHUNK, fill_zero, 0)

    row0 = s * TILE_ROWS
    def zero_acc(j, carry):
        pltpu.sync_copy(zbuf, acc.at[pl.ds(row0 + j * ZCHUNK, ZCHUNK)])
        return carry
    lax.fori_loop(0, TILE_ROWS // ZCHUNK, zero_acc, 0)
    plsc.subcore_barrier()

    def edge_pass(y_hbm):
        tile0 = s * SNBATCH
        def chunk(ci, carry):
            pltpu.sync_copy(src2.at[pl.ds(tile0 + ci * SCH, SCH)], sbuf)
            pltpu.sync_copy(dst2.at[pl.ds(tile0 + ci * SCH, SCH)], dbuf)
            g, sc = {}, {}
            for j in range(SCH):
                b = j & 1
                if j >= 2:
                    sc[j - 2].wait()
                g[j] = pltpu.async_copy(y_hbm.at[sbuf.at[j]], rows[b], gsems[b])
                if j >= 1:
                    g[j - 1].wait()
                    sc[j - 1] = pltpu.async_copy(
                        rows[(j - 1) & 1], acc.at[dbuf.at[j - 1]],
                        ssems[(j - 1) & 1], add=True)
            g[SCH - 1].wait()
            sc[SCH - 1] = pltpu.async_copy(
                rows[(SCH - 1) & 1], acc.at[dbuf.at[SCH - 1]],
                ssems[(SCH - 1) & 1], add=True)
            sc[SCH - 2].wait()
            sc[SCH - 1].wait()
            return carry
        lax.fori_loop(0, SNCH, chunk, 0)

    pl.when(c == 0)(lambda: edge_pass(ya))
    pl.when(c == 1)(lambda: edge_pass(yb))
    plsc.subcore_barrier()

    pl.when(c == 0)(lambda: pltpu.sync_copy(acc.at[pl.ds(row0, TILE_ROWS)],
                                            out_a.at[pl.ds(row0, TILE_ROWS)]))
    pl.when(c == 1)(lambda: pltpu.sync_copy(acc.at[pl.ds(row0, TILE_ROWS)],
                                            out_b.at[pl.ds(row0, TILE_ROWS)]))


def _dense0_body(f_ref, wred_ref, bred_ref, w0_ref, b0_ref, ae_ref, ya_ref, yb_ref):
    ae = jnp.dot(f_ref[...], wred_ref[...],
                 preferred_element_type=jnp.float32) + bred_ref[...]
    y0 = jnp.dot(ae, w0_ref[...], preferred_element_type=jnp.float32) + b0_ref[...]
    ae_ref[...] = ae
    ya_ref[...] = y0[:, :HH]
    yb_ref[...] = y0[:, HH:]


def _dense0(features, W_red, b_red, W0, b0):
    return pl.pallas_call(
        _dense0_body,
        grid=(N // ROW_BLK,),
        in_specs=[
            pl.BlockSpec((ROW_BLK, D_IN), lambda i: (i, 0)),
            pl.BlockSpec((D_IN, H_DIM), lambda i: (0, 0)),
            pl.BlockSpec((1, H_DIM), lambda i: (0, 0)),
            pl.BlockSpec((H_DIM, H_DIM), lambda i: (0, 0)),
            pl.BlockSpec((1, H_DIM), lambda i: (0, 0)),
        ],
        out_specs=[
            pl.BlockSpec((ROW_BLK, H_DIM), lambda i: (i, 0)),
            pl.BlockSpec((ROW_BLK, HH), lambda i: (i, 0)),
            pl.BlockSpec((ROW_BLK, HH), lambda i: (i, 0)),
        ],
        out_shape=[
            jax.ShapeDtypeStruct((N, H_DIM), jnp.float32),
            jax.ShapeDtypeStruct((N, HH), jnp.float32),
            jax.ShapeDtypeStruct((N, HH), jnp.float32),
        ],
    )(features, W_red, b_red.reshape(1, -1), W0, b0.reshape(1, -1))


def _dense1_body(sa_ref, sb_ref, deg_ref, w1_ref, b1_ref, x1_ref, ya_ref, yb_ref):
    inv = 1.0 / jnp.maximum(deg_ref[:, 0:1], 1.0)
    x1 = jnp.concatenate([sa_ref[...] * inv, sb_ref[...] * inv], axis=1)
    y1 = jnp.dot(x1, w1_ref[...], preferred_element_type=jnp.float32) + b1_ref[...]
    x1_ref[...] = x1
    ya_ref[...] = y1[:, :HH]
    yb_ref[...] = y1[:, HH:]


def _dense1(s0a, s0b, deg16, W1, b1):
    return pl.pallas_call(
        _dense1_body,
        grid=(N // ROW_BLK,),
        in_specs=[
            pl.BlockSpec((ROW_BLK, HH), lambda i: (i, 0)),
            pl.BlockSpec((ROW_BLK, HH), lambda i: (i, 0)),
            pl.BlockSpec((ROW_BLK, 16), lambda i: (i, 0)),
            pl.BlockSpec((H_DIM, H_DIM), lambda i: (0, 0)),
            pl.BlockSpec((1, H_DIM), lambda i: (0, 0)),
        ],
        out_specs=[
            pl.BlockSpec((ROW_BLK, H_DIM), lambda i: (i, 0)),
            pl.BlockSpec((ROW_BLK, HH), lambda i: (i, 0)),
            pl.BlockSpec((ROW_BLK, HH), lambda i: (i, 0)),
        ],
        out_shape=[
            jax.ShapeDtypeStruct((N, H_DIM), jnp.float32),
            jax.ShapeDtypeStruct((N, HH), jnp.float32),
            jax.ShapeDtypeStruct((N, HH), jnp.float32),
        ],
    )(s0a, s0b, deg16, W1, b1.reshape(1, -1))


def _final_body(ae_ref, x1_ref, sa_ref, sb_ref, deg_ref, wc_ref, bc_ref, out_ref):
    inv = 1.0 / jnp.maximum(deg_ref[:, 0:1], 1.0)
    x2 = jnp.concatenate([sa_ref[...] * inv, sb_ref[...] * inv], axis=1)
    m = (ae_ref[...] + x1_ref[...] + x2) * (1.0 / 3.0)
    z = jnp.dot(m, wc_ref[...], preferred_element_type=jnp.float32) + bc_ref[...]
    zmax = jnp.max(z, axis=1, keepdims=True)
    lse = jnp.log(jnp.sum(jnp.exp(z - zmax), axis=1, keepdims=True)) + zmax
    out_ref[...] = z - lse


def _final(all_emb, x1, s1a, s1b, deg16, W_cls, b_cls):
    return pl.pallas_call(
        _final_body,
        grid=(N_OUT // ROW_BLK,),
        in_specs=[
            pl.BlockSpec((ROW_BLK, H_DIM), lambda i: (i, 0)),
            pl.BlockSpec((ROW_BLK, H_DIM), lambda i: (i, 0)),
            pl.BlockSpec((ROW_BLK, HH), lambda i: (i, 0)),
            pl.BlockSpec((ROW_BLK, HH), lambda i: (i, 0)),
            pl.BlockSpec((ROW_BLK, 16), lambda i: (i, 0)),
            pl.BlockSpec((H_DIM, C_CLS), lambda i: (0, 0)),
            pl.BlockSpec((1, C_CLS), lambda i: (0, 0)),
        ],
        out_specs=pl.BlockSpec((ROW_BLK, C_CLS), lambda i: (i, 0)),
        out_shape=jax.ShapeDtypeStruct((N_OUT, C_CLS), jnp.float32),
    )(all_emb, x1, s1a, s1b, deg16, W_cls, b_cls.reshape(1, -1))


def kernel(features, edge_index, W_red, b_red, conv_weight_0, conv_bias_0,
           conv_weight_1, conv_bias_1, W_cls, b_cls):
    npad_e = EPAD - E_EDGES
    pad_src = jnp.zeros((npad_e,), jnp.int32)
    pad_dst = N + (jnp.arange(npad_e, dtype=jnp.int32) % (NPAD - N))
    src_p = jnp.concatenate([edge_index[0], pad_src])
    dst_p = jnp.concatenate([edge_index[1], pad_dst])
    dst2d = dst_p.reshape(EPAD // EB, EB)
    src2s = src_p.reshape(EPAD // SEB, SEB)
    dst2s = dst_p.reshape(EPAD // SEB, SEB)
    deg16 = _deg_sc(dst2d)
    all_emb, y0a, y0b = _dense0(features, W_red, b_red, conv_weight_0, conv_bias_0)
    s0a, s0b = _spmm_sc(y0a, y0b, src2s, dst2s)
    x1, y1a, y1b = _dense1(s0a, s0b, deg16, conv_weight_1, conv_bias_1)
    s1a, s1b = _spmm_sc(y1a, y1b, src2s, dst2s)
    return _final(all_emb, x1, s1a, s1b, deg16, W_cls, b_cls)


# speedup vs baseline: 1.2377x; 1.2321x over previous
import functools

import jax
import jax.numpy as jnp
from jax import lax
from jax.experimental import pallas as pl
from jax.experimental.pallas import tpu as pltpu
from jax.experimental.pallas import tpu_sc as plsc

N = 50000
N_OUT = 40000
E_EDGES = 800000
D_IN = 128
H_DIM = 64
HH = 32
C_CLS = 50

ROW_BLK = 1000
N_SUBC = 16
NPAD = 50048
TILE_ROWS = NPAD // N_SUBC
ZCHUNK = 184
EB = 80
EPAD = 819200
NBATCH = EPAD // EB // N_SUBC
G = 8
NG = NBATCH // G
NBUF = 4
SEB = 80
SNBATCH = E_EDGES // SEB // N_SUBC
SCH = 25
SNCH = SNBATCH // SCH

_sc_mesh = plsc.VectorSubcoreMesh(core_axis_name="c", subcore_axis_name="s")


@functools.partial(
    pl.kernel,
    mesh=_sc_mesh,
    compiler_params=pltpu.CompilerParams(use_tc_tiling_on_sc=False),
    out_type=jax.ShapeDtypeStruct((NPAD, 16), jnp.float32),
    scratch_types=[
        pltpu.VMEM((NBATCH, EB), jnp.int32),
        pltpu.VMEM((EB, 16), jnp.float32),
        pltpu.VMEM((ZCHUNK, 16), jnp.float32),
        pltpu.VMEM_SHARED((NPAD, 16), jnp.float32),
        pltpu.SemaphoreType.DMA,
        pltpu.SemaphoreType.DMA,
        pltpu.SemaphoreType.DMA,
        pltpu.SemaphoreType.DMA,
    ],
)
def _deg_sc(dst2, out_deg, dbuf, ones_v, zbuf, acc, sem0, sem1, sem2, sem3):
    c = lax.axis_index("c")
    s = lax.axis_index("s")
    sems = (sem0, sem1, sem2, sem3)

    @pl.when(c == 0)
    def _():
        def fill_ones(i, carry):
            ones_v[i, :] = jnp.ones((16,), jnp.float32)
            return carry
        lax.fori_loop(0, EB, fill_ones, 0)

        def fill_zero(i, carry):
            zbuf[i, :] = jnp.zeros((16,), jnp.float32)
            return carry
        lax.fori_loop(0, ZCHUNK, fill_zero, 0)

        row0 = s * TILE_ROWS
        def zero_acc(j, carry):
            pltpu.sync_copy(zbuf, acc.at[pl.ds(row0 + j * ZCHUNK, ZCHUNK)])
            return carry
        lax.fori_loop(0, TILE_ROWS // ZCHUNK, zero_acc, 0)
        pltpu.sync_copy(dst2.at[pl.ds(s * NBATCH, NBATCH)], dbuf)
        plsc.subcore_barrier()

        def ws_recon(b):
            pltpu.make_async_copy(ones_v, acc.at[dbuf.at[0]], sems[b]).wait()

        def group(gi, carry):
            base = gi * G
            sc = {}
            for j in range(G):
                b = j % NBUF
                if j < NBUF:
                    pl.when(gi > 0)(functools.partial(ws_recon, b))
                else:
                    sc[j - NBUF].wait()
                sc[j] = pltpu.async_copy(ones_v, acc.at[dbuf.at[base + j]],
                                         sems[b], add=True)
            return carry
        lax.fori_loop(0, NG, group, 0)
        for b in range(NBUF):
            ws_recon(b)
        plsc.subcore_barrier()

        pltpu.sync_copy(acc.at[pl.ds(row0, TILE_ROWS)],
                        out_deg.at[pl.ds(row0, TILE_ROWS)])


@functools.partial(
    pl.kernel,
    mesh=_sc_mesh,
    compiler_params=pltpu.CompilerParams(use_tc_tiling_on_sc=False),
    out_type=[jax.ShapeDtypeStruct((NPAD, HH), jnp.float32),
              jax.ShapeDtypeStruct((NPAD, HH), jnp.float32)],
    scratch_types=[
        pltpu.VMEM((SCH, SEB), jnp.int32),
        pltpu.VMEM((SCH, SEB), jnp.int32),
        pltpu.VMEM((SEB, HH), jnp.float32),
        pltpu.VMEM((SEB, HH), jnp.float32),
        pltpu.VMEM((ZCHUNK, HH), jnp.float32),
        pltpu.VMEM_SHARED((NPAD, HH), jnp.float32),
        pltpu.SemaphoreType.DMA,
        pltpu.SemaphoreType.DMA,
        pltpu.SemaphoreType.DMA,
        pltpu.SemaphoreType.DMA,
    ],
)
def _spmm_sc(ya, yb, src2, dst2, out_a, out_b, sbuf, dbuf,
             rows0, rows1, zbuf, acc, gsem0, gsem1, ssem0, ssem1):
    c = lax.axis_index("c")
    s = lax.axis_index("s")
    rows = (rows0, rows1)
    gsems = (gsem0, gsem1)
    ssems = (ssem0, ssem1)

    def fill_zero(i, carry):
        zbuf[i, pl.ds(0, 16)] = jnp.zeros((16,), jnp.float32)
        zbuf[i, pl.ds(16, 16)] = jnp.zeros((16,), jnp.float32)
        return carry
    lax.fori_loop(0, ZCHUNK, fill_zero, 0)

    row0 = s * TILE_ROWS
    def zero_acc(j, carry):
        pltpu.sync_copy(zbuf, acc.at[pl.ds(row0 + j * ZCHUNK, ZCHUNK)])
        return carry
    lax.fori_loop(0, TILE_ROWS // ZCHUNK, zero_acc, 0)
    plsc.subcore_barrier()

    def edge_pass(y_hbm):
        tile0 = s * SNBATCH
        def chunk(ci, carry):
            pltpu.sync_copy(src2.at[pl.ds(tile0 + ci * SCH, SCH)], sbuf)
            pltpu.sync_copy(dst2.at[pl.ds(tile0 + ci * SCH, SCH)], dbuf)
            g, sc = {}, {}
            for j in range(SCH):
                b = j & 1
                if j >= 2:
                    sc[j - 2].wait()
                g[j] = pltpu.async_copy(y_hbm.at[sbuf.at[j]], rows[b], gsems[b])
                if j >= 1:
                    g[j - 1].wait()
                    sc[j - 1] = pltpu.async_copy(
                        rows[(j - 1) & 1], acc.at[dbuf.at[j - 1]],
                        ssems[(j - 1) & 1], add=True)
            g[SCH - 1].wait()
            sc[SCH - 1] = pltpu.async_copy(
                rows[(SCH - 1) & 1], acc.at[dbuf.at[SCH - 1]],
                ssems[(SCH - 1) & 1], add=True)
            sc[SCH - 2].wait()
            sc[SCH - 1].wait()
            return carry
        lax.fori_loop(0, SNCH, chunk, 0)

    pl.when(c == 0)(lambda: edge_pass(ya))
    pl.when(c == 1)(lambda: edge_pass(yb))
    plsc.subcore_barrier()

    pl.when(c == 0)(lambda: pltpu.sync_copy(acc.at[pl.ds(row0, TILE_ROWS)],
                                            out_a.at[pl.ds(row0, TILE_ROWS)]))
    pl.when(c == 1)(lambda: pltpu.sync_copy(acc.at[pl.ds(row0, TILE_ROWS)],
                                            out_b.at[pl.ds(row0, TILE_ROWS)]))


def _dense0_body(f_ref, wred_ref, bred_ref, w0_ref, b0_ref, ae_ref, ya_ref, yb_ref):
    ae = jnp.dot(f_ref[...], wred_ref[...],
                 preferred_element_type=jnp.float32) + bred_ref[...]
    y0 = jnp.dot(ae, w0_ref[...], preferred_element_type=jnp.float32) + b0_ref[...]
    ae_ref[...] = ae
    ya_ref[...] = y0[:, :HH]
    yb_ref[...] = y0[:, HH:]


def _dense0(features, W_red, b_red, W0, b0):
    return pl.pallas_call(
        _dense0_body,
        grid=(N // ROW_BLK,),
        in_specs=[
            pl.BlockSpec((ROW_BLK, D_IN), lambda i: (i, 0)),
            pl.BlockSpec((D_IN, H_DIM), lambda i: (0, 0)),
            pl.BlockSpec((1, H_DIM), lambda i: (0, 0)),
            pl.BlockSpec((H_DIM, H_DIM), lambda i: (0, 0)),
            pl.BlockSpec((1, H_DIM), lambda i: (0, 0)),
        ],
        out_specs=[
            pl.BlockSpec((ROW_BLK, H_DIM), lambda i: (i, 0)),
            pl.BlockSpec((ROW_BLK, HH), lambda i: (i, 0)),
            pl.BlockSpec((ROW_BLK, HH), lambda i: (i, 0)),
        ],
        out_shape=[
            jax.ShapeDtypeStruct((N, H_DIM), jnp.float32),
            jax.ShapeDtypeStruct((N, HH), jnp.float32),
            jax.ShapeDtypeStruct((N, HH), jnp.float32),
        ],
    )(features, W_red, b_red.reshape(1, -1), W0, b0.reshape(1, -1))


def _dense1_body(sa_ref, sb_ref, deg_ref, w1_ref, b1_ref, x1_ref, ya_ref, yb_ref):
    inv = 1.0 / jnp.maximum(deg_ref[:, 0:1], 1.0)
    x1 = jnp.concatenate([sa_ref[...] * inv, sb_ref[...] * inv], axis=1)
    y1 = jnp.dot(x1, w1_ref[...], preferred_element_type=jnp.float32) + b1_ref[...]
    x1_ref[...] = x1
    ya_ref[...] = y1[:, :HH]
    yb_ref[...] = y1[:, HH:]


def _dense1(s0a, s0b, deg16, W1, b1):
    return pl.pallas_call(
        _dense1_body,
        grid=(N // ROW_BLK,),
        in_specs=[
            pl.BlockSpec((ROW_BLK, HH), lambda i: (i, 0)),
            pl.BlockSpec((ROW_BLK, HH), lambda i: (i, 0)),
            pl.BlockSpec((ROW_BLK, 16), lambda i: (i, 0)),
            pl.BlockSpec((H_DIM, H_DIM), lambda i: (0, 0)),
            pl.BlockSpec((1, H_DIM), lambda i: (0, 0)),
        ],
        out_specs=[
            pl.BlockSpec((ROW_BLK, H_DIM), lambda i: (i, 0)),
            pl.BlockSpec((ROW_BLK, HH), lambda i: (i, 0)),
            pl.BlockSpec((ROW_BLK, HH), lambda i: (i, 0)),
        ],
        out_shape=[
            jax.ShapeDtypeStruct((N, H_DIM), jnp.float32),
            jax.ShapeDtypeStruct((N, HH), jnp.float32),
            jax.ShapeDtypeStruct((N, HH), jnp.float32),
        ],
    )(s0a, s0b, deg16, W1, b1.reshape(1, -1))


def _final_body(ae_ref, x1_ref, sa_ref, sb_ref, deg_ref, wc_ref, bc_ref, out_ref):
    inv = 1.0 / jnp.maximum(deg_ref[:, 0:1], 1.0)
    x2 = jnp.concatenate([sa_ref[...] * inv, sb_ref[...] * inv], axis=1)
    m = (ae_ref[...] + x1_ref[...] + x2) * (1.0 / 3.0)
    z = jnp.dot(m, wc_ref[...], preferred_element_type=jnp.float32) + bc_ref[...]
    zmax = jnp.max(z, axis=1, keepdims=True)
    lse = jnp.log(jnp.sum(jnp.exp(z - zmax), axis=1, keepdims=True)) + zmax
    out_ref[...] = z - lse


def _final(all_emb, x1, s1a, s1b, deg16, W_cls, b_cls):
    return pl.pallas_call(
        _final_body,
        grid=(N_OUT // ROW_BLK,),
        in_specs=[
            pl.BlockSpec((ROW_BLK, H_DIM), lambda i: (i, 0)),
            pl.BlockSpec((ROW_BLK, H_DIM), lambda i: (i, 0)),
            pl.BlockSpec((ROW_BLK, HH), lambda i: (i, 0)),
            pl.BlockSpec((ROW_BLK, HH), lambda i: (i, 0)),
            pl.BlockSpec((ROW_BLK, 16), lambda i: (i, 0)),
            pl.BlockSpec((H_DIM, C_CLS), lambda i: (0, 0)),
            pl.BlockSpec((1, C_CLS), lambda i: (0, 0)),
        ],
        out_specs=pl.BlockSpec((ROW_BLK, C_CLS), lambda i: (i, 0)),
        out_shape=jax.ShapeDtypeStruct((N_OUT, C_CLS), jnp.float32),
    )(all_emb, x1, s1a, s1b, deg16, W_cls, b_cls.reshape(1, -1))


def kernel(features, edge_index, W_red, b_red, conv_weight_0, conv_bias_0,
           conv_weight_1, conv_bias_1, W_cls, b_cls):
    npad_e = EPAD - E_EDGES
    pad_src = jnp.zeros((npad_e,), jnp.int32)
    pad_dst = N + (jnp.arange(npad_e, dtype=jnp.int32) % (NPAD - N))
    src_p = jnp.concatenate([edge_index[0], pad_src])
    dst_p = jnp.concatenate([edge_index[1], pad_dst])
    dst2d = dst_p.reshape(EPAD // EB, EB)
    src2s = edge_index[0].reshape(E_EDGES // SEB, SEB)
    dst2s = edge_index[1].reshape(E_EDGES // SEB, SEB)
    deg16 = _deg_sc(dst2d)
    all_emb, y0a, y0b = _dense0(features, W_red, b_red, conv_weight_0, conv_bias_0)
    s0a, s0b = _spmm_sc(y0a, y0b, src2s, dst2s)
    x1, y1a, y1b = _dense1(s0a, s0b, deg16, conv_weight_1, conv_bias_1)
    s1a, s1b = _spmm_sc(y1a, y1b, src2s, dst2s)
    return _final(all_emb, x1, s1a, s1b, deg16, W_cls, b_cls)


# speedup vs baseline: 1.2942x; 1.0456x over previous
import functools

import jax
import jax.numpy as jnp
from jax import lax
from jax.experimental import pallas as pl
from jax.experimental.pallas import tpu as pltpu
from jax.experimental.pallas import tpu_sc as plsc

N = 50000
N_OUT = 40000
E_EDGES = 800000
D_IN = 128
H_DIM = 64
HH = 32
C_CLS = 50

ROW_BLK = 2000
N_SUBC = 16
NPAD = 50048
TILE_ROWS = NPAD // N_SUBC
ZCHUNK = 184
EB = 80
NBATCH = E_EDGES // EB // N_SUBC
G = 25
NG = NBATCH // G
NBUF = 5
SEB = 80
SNBATCH = E_EDGES // SEB // N_SUBC
SCH = 25
SNCH = SNBATCH // SCH

_sc_mesh = plsc.VectorSubcoreMesh(core_axis_name="c", subcore_axis_name="s")


@functools.partial(
    pl.kernel,
    mesh=_sc_mesh,
    compiler_params=pltpu.CompilerParams(use_tc_tiling_on_sc=False),
    out_type=jax.ShapeDtypeStruct((NPAD, 16), jnp.float32),
    scratch_types=[
        pltpu.VMEM((NBATCH, EB), jnp.int32),
        pltpu.VMEM((EB, 16), jnp.float32),
        pltpu.VMEM((ZCHUNK, 16), jnp.float32),
        pltpu.VMEM_SHARED((NPAD, 16), jnp.float32),
        pltpu.SemaphoreType.DMA,
        pltpu.SemaphoreType.DMA,
        pltpu.SemaphoreType.DMA,
        pltpu.SemaphoreType.DMA,
        pltpu.SemaphoreType.DMA,
    ],
)
def _deg_sc(dst2, out_deg, dbuf, ones_v, zbuf, acc, sem0, sem1, sem2, sem3, sem4):
    c = lax.axis_index("c")
    s = lax.axis_index("s")
    sems = (sem0, sem1, sem2, sem3, sem4)

    @pl.when(c == 0)
    def _():
        def fill_ones(i, carry):
            ones_v[i, :] = jnp.ones((16,), jnp.float32)
            return carry
        lax.fori_loop(0, EB, fill_ones, 0)

        def fill_zero(i, carry):
            zbuf[i, :] = jnp.zeros((16,), jnp.float32)
            return carry
        lax.fori_loop(0, ZCHUNK, fill_zero, 0)

        row0 = s * TILE_ROWS
        def zero_acc(j, carry):
            pltpu.sync_copy(zbuf, acc.at[pl.ds(row0 + j * ZCHUNK, ZCHUNK)])
            return carry
        lax.fori_loop(0, TILE_ROWS // ZCHUNK, zero_acc, 0)
        pltpu.sync_copy(dst2.at[pl.ds(s * NBATCH, NBATCH)], dbuf)
        plsc.subcore_barrier()

        def ws_recon(b):
            pltpu.make_async_copy(ones_v, acc.at[dbuf.at[0]], sems[b]).wait()

        def group(gi, carry):
            base = gi * G
            sc = {}
            for j in range(G):
                b = j % NBUF
                if j < NBUF:
                    pl.when(gi > 0)(functools.partial(ws_recon, b))
                else:
                    sc[j - NBUF].wait()
                sc[j] = pltpu.async_copy(ones_v, acc.at[dbuf.at[base + j]],
                                         sems[b], add=True)
            return carry
        lax.fori_loop(0, NG, group, 0)
        for b in range(NBUF):
            ws_recon(b)
        plsc.subcore_barrier()

        pltpu.sync_copy(acc.at[pl.ds(row0, TILE_ROWS)],
                        out_deg.at[pl.ds(row0, TILE_ROWS)])


@functools.partial(
    pl.kernel,
    mesh=_sc_mesh,
    compiler_params=pltpu.CompilerParams(use_tc_tiling_on_sc=False),
    out_type=[jax.ShapeDtypeStruct((NPAD, HH), jnp.float32),
              jax.ShapeDtypeStruct((NPAD, HH), jnp.float32)],
    scratch_types=[
        pltpu.VMEM((SCH, SEB), jnp.int32),
        pltpu.VMEM((SCH, SEB), jnp.int32),
        pltpu.VMEM((SEB, HH), jnp.float32),
        pltpu.VMEM((SEB, HH), jnp.float32),
        pltpu.VMEM((ZCHUNK, HH), jnp.float32),
        pltpu.VMEM_SHARED((NPAD, HH), jnp.float32),
        pltpu.SemaphoreType.DMA,
        pltpu.SemaphoreType.DMA,
        pltpu.SemaphoreType.DMA,
        pltpu.SemaphoreType.DMA,
    ],
)
def _spmm_sc(ya, yb, src2, dst2, out_a, out_b, sbuf, dbuf,
             rows0, rows1, zbuf, acc, gsem0, gsem1, ssem0, ssem1):
    c = lax.axis_index("c")
    s = lax.axis_index("s")
    rows = (rows0, rows1)
    gsems = (gsem0, gsem1)
    ssems = (ssem0, ssem1)

    def fill_zero(i, carry):
        zbuf[i, pl.ds(0, 16)] = jnp.zeros((16,), jnp.float32)
        zbuf[i, pl.ds(16, 16)] = jnp.zeros((16,), jnp.float32)
        return carry
    lax.fori_loop(0, ZCHUNK, fill_zero, 0)

    row0 = s * TILE_ROWS
    def zero_acc(j, carry):
        pltpu.sync_copy(zbuf, acc.at[pl.ds(row0 + j * ZCHUNK, ZCHUNK)])
        return carry
    lax.fori_loop(0, TILE_ROWS // ZCHUNK, zero_acc, 0)
    plsc.subcore_barrier()

    def edge_pass(y_hbm):
        tile0 = s * SNBATCH
        def chunk(ci, carry):
            pltpu.sync_copy(src2.at[pl.ds(tile0 + ci * SCH, SCH)], sbuf)
            pltpu.sync_copy(dst2.at[pl.ds(tile0 + ci * SCH, SCH)], dbuf)
            g, sc = {}, {}
            for j in range(SCH):
                b = j & 1
                if j >= 2:
                    sc[j - 2].wait()
                g[j] = pltpu.async_copy(y_hbm.at[sbuf.at[j]], rows[b], gsems[b])
                if j >= 1:
                    g[j - 1].wait()
                    sc[j - 1] = pltpu.async_copy(
                        rows[(j - 1) & 1], acc.at[dbuf.at[j - 1]],
                        ssems[(j - 1) & 1], add=True)
            g[SCH - 1].wait()
            sc[SCH - 1] = pltpu.async_copy(
                rows[(SCH - 1) & 1], acc.at[dbuf.at[SCH - 1]],
                ssems[(SCH - 1) & 1], add=True)
            sc[SCH - 2].wait()
            sc[SCH - 1].wait()
            return carry
        lax.fori_loop(0, SNCH, chunk, 0)

    pl.when(c == 0)(lambda: edge_pass(ya))
    pl.when(c == 1)(lambda: edge_pass(yb))
    plsc.subcore_barrier()

    pl.when(c == 0)(lambda: pltpu.sync_copy(acc.at[pl.ds(row0, TILE_ROWS)],
                                            out_a.at[pl.ds(row0, TILE_ROWS)]))
    pl.when(c == 1)(lambda: pltpu.sync_copy(acc.at[pl.ds(row0, TILE_ROWS)],
                                            out_b.at[pl.ds(row0, TILE_ROWS)]))


def _dense0_body(f_ref, wred_ref, bred_ref, w0_ref, b0_ref, ae_ref, ya_ref, yb_ref):
    ae = jnp.dot(f_ref[...], wred_ref[...],
                 preferred_element_type=jnp.float32) + bred_ref[...]
    y0 = jnp.dot(ae, w0_ref[...], preferred_element_type=jnp.float32) + b0_ref[...]
    ae_ref[...] = ae
    ya_ref[...] = y0[:, :HH]
    yb_ref[...] = y0[:, HH:]


def _dense0(features, W_red, b_red, W0, b0):
    return pl.pallas_call(
        _dense0_body,
        grid=(N // ROW_BLK,),
        in_specs=[
            pl.BlockSpec((ROW_BLK, D_IN), lambda i: (i, 0)),
            pl.BlockSpec((D_IN, H_DIM), lambda i: (0, 0)),
            pl.BlockSpec((1, H_DIM), lambda i: (0, 0)),
            pl.BlockSpec((H_DIM, H_DIM), lambda i: (0, 0)),
            pl.BlockSpec((1, H_DIM), lambda i: (0, 0)),
        ],
        out_specs=[
            pl.BlockSpec((ROW_BLK, H_DIM), lambda i: (i, 0)),
            pl.BlockSpec((ROW_BLK, HH), lambda i: (i, 0)),
            pl.BlockSpec((ROW_BLK, HH), lambda i: (i, 0)),
        ],
        out_shape=[
            jax.ShapeDtypeStruct((N, H_DIM), jnp.float32),
            jax.ShapeDtypeStruct((N, HH), jnp.float32),
            jax.ShapeDtypeStruct((N, HH), jnp.float32),
        ],
    )(features, W_red, b_red.reshape(1, -1), W0, b0.reshape(1, -1))


def _dense1_body(sa_ref, sb_ref, deg_ref, w1_ref, b1_ref, x1_ref, ya_ref, yb_ref):
    inv = 1.0 / jnp.maximum(deg_ref[:, 0:1], 1.0)
    x1 = jnp.concatenate([sa_ref[...] * inv, sb_ref[...] * inv], axis=1)
    y1 = jnp.dot(x1, w1_ref[...], preferred_element_type=jnp.float32) + b1_ref[...]
    x1_ref[...] = x1
    ya_ref[...] = y1[:, :HH]
    yb_ref[...] = y1[:, HH:]


def _dense1(s0a, s0b, deg16, W1, b1):
    return pl.pallas_call(
        _dense1_body,
        grid=(N // ROW_BLK,),
        in_specs=[
            pl.BlockSpec((ROW_BLK, HH), lambda i: (i, 0)),
            pl.BlockSpec((ROW_BLK, HH), lambda i: (i, 0)),
            pl.BlockSpec((ROW_BLK, 16), lambda i: (i, 0)),
            pl.BlockSpec((H_DIM, H_DIM), lambda i: (0, 0)),
            pl.BlockSpec((1, H_DIM), lambda i: (0, 0)),
        ],
        out_specs=[
            pl.BlockSpec((ROW_BLK, H_DIM), lambda i: (i, 0)),
            pl.BlockSpec((ROW_BLK, HH), lambda i: (i, 0)),
            pl.BlockSpec((ROW_BLK, HH), lambda i: (i, 0)),
        ],
        out_shape=[
            jax.ShapeDtypeStruct((N, H_DIM), jnp.float32),
            jax.ShapeDtypeStruct((N, HH), jnp.float32),
            jax.ShapeDtypeStruct((N, HH), jnp.float32),
        ],
    )(s0a, s0b, deg16, W1, b1.reshape(1, -1))


def _final_body(ae_ref, x1_ref, sa_ref, sb_ref, deg_ref, wc_ref, bc_ref, out_ref):
    inv = 1.0 / jnp.maximum(deg_ref[:, 0:1], 1.0)
    x2 = jnp.concatenate([sa_ref[...] * inv, sb_ref[...] * inv], axis=1)
    m = (ae_ref[...] + x1_ref[...] + x2) * (1.0 / 3.0)
    z = jnp.dot(m, wc_ref[...], preferred_element_type=jnp.float32) + bc_ref[...]
    zmax = jnp.max(z, axis=1, keepdims=True)
    lse = jnp.log(jnp.sum(jnp.exp(z - zmax), axis=1, keepdims=True)) + zmax
    out_ref[...] = z - lse


def _final(all_emb, x1, s1a, s1b, deg16, W_cls, b_cls):
    return pl.pallas_call(
        _final_body,
        grid=(N_OUT // ROW_BLK,),
        in_specs=[
            pl.BlockSpec((ROW_BLK, H_DIM), lambda i: (i, 0)),
            pl.BlockSpec((ROW_BLK, H_DIM), lambda i: (i, 0)),
            pl.BlockSpec((ROW_BLK, HH), lambda i: (i, 0)),
            pl.BlockSpec((ROW_BLK, HH), lambda i: (i, 0)),
            pl.BlockSpec((ROW_BLK, 16), lambda i: (i, 0)),
            pl.BlockSpec((H_DIM, C_CLS), lambda i: (0, 0)),
            pl.BlockSpec((1, C_CLS), lambda i: (0, 0)),
        ],
        out_specs=pl.BlockSpec((ROW_BLK, C_CLS), lambda i: (i, 0)),
        out_shape=jax.ShapeDtypeStruct((N_OUT, C_CLS), jnp.float32),
    )(all_emb, x1, s1a, s1b, deg16, W_cls, b_cls.reshape(1, -1))


def kernel(features, edge_index, W_red, b_red, conv_weight_0, conv_bias_0,
           conv_weight_1, conv_bias_1, W_cls, b_cls):
    dst2d = edge_index[1].reshape(E_EDGES // EB, EB)
    src2s = edge_index[0].reshape(E_EDGES // SEB, SEB)
    dst2s = edge_index[1].reshape(E_EDGES // SEB, SEB)
    deg16 = _deg_sc(dst2d)
    all_emb, y0a, y0b = _dense0(features, W_red, b_red, conv_weight_0, conv_bias_0)
    s0a, s0b = _spmm_sc(y0a, y0b, src2s, dst2s)
    x1, y1a, y1b = _dense1(s0a, s0b, deg16, conv_weight_1, conv_bias_1)
    s1a, s1b = _spmm_sc(y1a, y1b, src2s, dst2s)
    return _final(all_emb, x1, s1a, s1b, deg16, W_cls, b_cls)


# speedup vs baseline: 1.5378x; 1.1883x over previous
import functools

import jax
import jax.numpy as jnp
from jax import lax
from jax.experimental import pallas as pl
from jax.experimental.pallas import tpu as pltpu
from jax.experimental.pallas import tpu_sc as plsc

N = 50000
N_OUT = 40000
E_EDGES = 800000
D_IN = 128
H_DIM = 64
HH = 32
C_CLS = 50

ROW_BLK = 2000
N_SUBC = 16
NPAD = 50048
TILE_ROWS = NPAD // N_SUBC
ZCHUNK = 184
EB = 80
NBATCH = E_EDGES // EB // N_SUBC
G = 25
NG = NBATCH // G
NBUF = 5
SEB = 80
SNBATCH = E_EDGES // SEB // N_SUBC
SCH = 25
SNCH = SNBATCH // SCH

_sc_mesh = plsc.VectorSubcoreMesh(core_axis_name="c", subcore_axis_name="s")


@functools.partial(
    pl.kernel,
    mesh=_sc_mesh,
    compiler_params=pltpu.CompilerParams(use_tc_tiling_on_sc=False),
    out_type=jax.ShapeDtypeStruct((NPAD, 16), jnp.float32),
    scratch_types=[
        pltpu.VMEM((NBATCH, EB), jnp.int32),
        pltpu.VMEM((EB, 16), jnp.float32),
        pltpu.VMEM((ZCHUNK, 16), jnp.float32),
        pltpu.VMEM_SHARED((NPAD, 16), jnp.float32),
        pltpu.SemaphoreType.DMA,
        pltpu.SemaphoreType.DMA,
        pltpu.SemaphoreType.DMA,
        pltpu.SemaphoreType.DMA,
        pltpu.SemaphoreType.DMA,
    ],
)
def _deg_sc(dst2, out_deg, dbuf, ones_v, zbuf, acc, sem0, sem1, sem2, sem3, sem4):
    c = lax.axis_index("c")
    s = lax.axis_index("s")
    sems = (sem0, sem1, sem2, sem3, sem4)

    @pl.when(c == 0)
    def _():
        def fill_ones(i, carry):
            ones_v[i, :] = jnp.ones((16,), jnp.float32)
            return carry
        lax.fori_loop(0, EB, fill_ones, 0)

        def fill_zero(i, carry):
            zbuf[i, :] = jnp.zeros((16,), jnp.float32)
            return carry
        lax.fori_loop(0, ZCHUNK, fill_zero, 0)

        row0 = s * TILE_ROWS
        def zero_acc(j, carry):
            pltpu.sync_copy(zbuf, acc.at[pl.ds(row0 + j * ZCHUNK, ZCHUNK)])
            return carry
        lax.fori_loop(0, TILE_ROWS // ZCHUNK, zero_acc, 0)
        pltpu.sync_copy(dst2.at[pl.ds(s * NBATCH, NBATCH)], dbuf)
        plsc.subcore_barrier()

        def ws_recon(b):
            pltpu.make_async_copy(ones_v, acc.at[dbuf.at[0]], sems[b]).wait()

        def group(gi, carry):
            base = gi * G
            sc = {}
            for j in range(G):
                b = j % NBUF
                if j < NBUF:
                    pl.when(gi > 0)(functools.partial(ws_recon, b))
                else:
                    sc[j - NBUF].wait()
                sc[j] = pltpu.async_copy(ones_v, acc.at[dbuf.at[base + j]],
                                         sems[b], add=True)
            return carry
        lax.fori_loop(0, NG, group, 0)
        for b in range(NBUF):
            ws_recon(b)
        plsc.subcore_barrier()

        pltpu.sync_copy(acc.at[pl.ds(row0, TILE_ROWS)],
                        out_deg.at[pl.ds(row0, TILE_ROWS)])


@functools.partial(
    pl.kernel,
    mesh=_sc_mesh,
    compiler_params=pltpu.CompilerParams(use_tc_tiling_on_sc=False),
    out_type=[jax.ShapeDtypeStruct((NPAD, HH), jnp.float32),
              jax.ShapeDtypeStruct((NPAD, HH), jnp.float32)],
    scratch_types=[
        pltpu.VMEM((SCH, SEB), jnp.int32),
        pltpu.VMEM((SCH, SEB), jnp.int32),
        pltpu.VMEM((SEB, HH), jnp.float32),
        pltpu.VMEM((SEB, HH), jnp.float32),
        pltpu.VMEM((SEB, HH), jnp.float32),
        pltpu.VMEM((ZCHUNK, HH), jnp.float32),
        pltpu.VMEM_SHARED((NPAD, HH), jnp.float32),
        pltpu.SemaphoreType.DMA,
        pltpu.SemaphoreType.DMA,
        pltpu.SemaphoreType.DMA,
        pltpu.SemaphoreType.DMA,
        pltpu.SemaphoreType.DMA,
        pltpu.SemaphoreType.DMA,
    ],
)
def _spmm_sc(ya, yb, src2, dst2, out_a, out_b, sbuf, dbuf,
             rows0, rows1, rows2, zbuf, acc,
             gsem0, gsem1, gsem2, ssem0, ssem1, ssem2):
    c = lax.axis_index("c")
    s = lax.axis_index("s")
    rows = (rows0, rows1, rows2)
    gsems = (gsem0, gsem1, gsem2)
    ssems = (ssem0, ssem1, ssem2)

    def fill_zero(i, carry):
        zbuf[i, pl.ds(0, 16)] = jnp.zeros((16,), jnp.float32)
        zbuf[i, pl.ds(16, 16)] = jnp.zeros((16,), jnp.float32)
        return carry
    lax.fori_loop(0, ZCHUNK, fill_zero, 0)

    row0 = s * TILE_ROWS
    def zero_acc(j, carry):
        pltpu.sync_copy(zbuf, acc.at[pl.ds(row0 + j * ZCHUNK, ZCHUNK)])
        return carry
    lax.fori_loop(0, TILE_ROWS // ZCHUNK, zero_acc, 0)
    plsc.subcore_barrier()

    def edge_pass(y_hbm):
        tile0 = s * SNBATCH
        def chunk(ci, carry):
            pltpu.sync_copy(src2.at[pl.ds(tile0 + ci * SCH, SCH)], sbuf)
            pltpu.sync_copy(dst2.at[pl.ds(tile0 + ci * SCH, SCH)], dbuf)
            g, sc = {}, {}
            def isc(t):
                return pltpu.async_copy(rows[t % 3], acc.at[dbuf.at[t]],
                                        ssems[t % 3], add=True)
            for j in range(SCH):
                b = j % 3
                if j >= 3:
                    sc[j - 3].wait()
                g[j] = pltpu.async_copy(y_hbm.at[sbuf.at[j]], rows[b], gsems[b])
                if j >= 2:
                    g[j - 2].wait()
                    sc[j - 2] = isc(j - 2)
            for t in (SCH - 2, SCH - 1):
                g[t].wait()
                sc[t] = isc(t)
            sc[SCH - 3].wait()
            sc[SCH - 2].wait()
            sc[SCH - 1].wait()
            return carry
        lax.fori_loop(0, SNCH, chunk, 0)

    pl.when(c == 0)(lambda: edge_pass(ya))
    pl.when(c == 1)(lambda: edge_pass(yb))
    plsc.subcore_barrier()

    pl.when(c == 0)(lambda: pltpu.sync_copy(acc.at[pl.ds(row0, TILE_ROWS)],
                                            out_a.at[pl.ds(row0, TILE_ROWS)]))
    pl.when(c == 1)(lambda: pltpu.sync_copy(acc.at[pl.ds(row0, TILE_ROWS)],
                                            out_b.at[pl.ds(row0, TILE_ROWS)]))


def _dense0_body(f_ref, wred_ref, bred_ref, w0_ref, b0_ref, ae_ref, ya_ref, yb_ref):
    ae = jnp.dot(f_ref[...], wred_ref[...],
                 preferred_element_type=jnp.float32) + bred_ref[...]
    y0 = jnp.dot(ae, w0_ref[...], preferred_element_type=jnp.float32) + b0_ref[...]
    ae_ref[...] = ae
    ya_ref[...] = y0[:, :HH]
    yb_ref[...] = y0[:, HH:]


def _dense0(features, W_red, b_red, W0, b0):
    return pl.pallas_call(
        _dense0_body,
        grid=(N // ROW_BLK,),
        in_specs=[
            pl.BlockSpec((ROW_BLK, D_IN), lambda i: (i, 0)),
            pl.BlockSpec((D_IN, H_DIM), lambda i: (0, 0)),
            pl.BlockSpec((1, H_DIM), lambda i: (0, 0)),
            pl.BlockSpec((H_DIM, H_DIM), lambda i: (0, 0)),
            pl.BlockSpec((1, H_DIM), lambda i: (0, 0)),
        ],
        out_specs=[
            pl.BlockSpec((ROW_BLK, H_DIM), lambda i: (i, 0)),
            pl.BlockSpec((ROW_BLK, HH), lambda i: (i, 0)),
            pl.BlockSpec((ROW_BLK, HH), lambda i: (i, 0)),
        ],
        out_shape=[
            jax.ShapeDtypeStruct((N, H_DIM), jnp.float32),
            jax.ShapeDtypeStruct((N, HH), jnp.float32),
            jax.ShapeDtypeStruct((N, HH), jnp.float32),
        ],
    )(features, W_red, b_red.reshape(1, -1), W0, b0.reshape(1, -1))


def _dense1_body(sa_ref, sb_ref, deg_ref, w1_ref, b1_ref, x1_ref, ya_ref, yb_ref):
    inv = 1.0 / jnp.maximum(deg_ref[:, 0:1], 1.0)
    x1 = jnp.concatenate([sa_ref[...] * inv, sb_ref[...] * inv], axis=1)
    y1 = jnp.dot(x1, w1_ref[...], preferred_element_type=jnp.float32) + b1_ref[...]
    x1_ref[...] = x1
    ya_ref[...] = y1[:, :HH]
    yb_ref[...] = y1[:, HH:]


def _dense1(s0a, s0b, deg16, W1, b1):
    return pl.pallas_call(
        _dense1_body,
        grid=(N // ROW_BLK,),
        in_specs=[
            pl.BlockSpec((ROW_BLK, HH), lambda i: (i, 0)),
            pl.BlockSpec((ROW_BLK, HH), lambda i: (i, 0)),
            pl.BlockSpec((ROW_BLK, 16), lambda i: (i, 0)),
            pl.BlockSpec((H_DIM, H_DIM), lambda i: (0, 0)),
            pl.BlockSpec((1, H_DIM), lambda i: (0, 0)),
        ],
        out_specs=[
            pl.BlockSpec((ROW_BLK, H_DIM), lambda i: (i, 0)),
            pl.BlockSpec((ROW_BLK, HH), lambda i: (i, 0)),
            pl.BlockSpec((ROW_BLK, HH), lambda i: (i, 0)),
        ],
        out_shape=[
            jax.ShapeDtypeStruct((N, H_DIM), jnp.float32),
            jax.ShapeDtypeStruct((N, HH), jnp.float32),
            jax.ShapeDtypeStruct((N, HH), jnp.float32),
        ],
    )(s0a, s0b, deg16, W1, b1.reshape(1, -1))


def _final_body(ae_ref, x1_ref, sa_ref, sb_ref, deg_ref, wc_ref, bc_ref, out_ref):
    inv = 1.0 / jnp.maximum(deg_ref[:, 0:1], 1.0)
    x2 = jnp.concatenate([sa_ref[...] * inv, sb_ref[...] * inv], axis=1)
    m = (ae_ref[...] + x1_ref[...] + x2) * (1.0 / 3.0)
    z = jnp.dot(m, wc_ref[...], preferred_element_type=jnp.float32) + bc_ref[...]
    zmax = jnp.max(z, axis=1, keepdims=True)
    lse = jnp.log(jnp.sum(jnp.exp(z - zmax), axis=1, keepdims=True)) + zmax
    out_ref[...] = z - lse


def _final(all_emb, x1, s1a, s1b, deg16, W_cls, b_cls):
    return pl.pallas_call(
        _final_body,
        grid=(N_OUT // ROW_BLK,),
        in_specs=[
            pl.BlockSpec((ROW_BLK, H_DIM), lambda i: (i, 0)),
            pl.BlockSpec((ROW_BLK, H_DIM), lambda i: (i, 0)),
            pl.BlockSpec((ROW_BLK, HH), lambda i: (i, 0)),
            pl.BlockSpec((ROW_BLK, HH), lambda i: (i, 0)),
            pl.BlockSpec((ROW_BLK, 16), lambda i: (i, 0)),
            pl.BlockSpec((H_DIM, C_CLS), lambda i: (0, 0)),
            pl.BlockSpec((1, C_CLS), lambda i: (0, 0)),
        ],
        out_specs=pl.BlockSpec((ROW_BLK, C_CLS), lambda i: (i, 0)),
        out_shape=jax.ShapeDtypeStruct((N_OUT, C_CLS), jnp.float32),
    )(all_emb, x1, s1a, s1b, deg16, W_cls, b_cls.reshape(1, -1))


def kernel(features, edge_index, W_red, b_red, conv_weight_0, conv_bias_0,
           conv_weight_1, conv_bias_1, W_cls, b_cls):
    dst2d = edge_index[1].reshape(E_EDGES // EB, EB)
    src2s = edge_index[0].reshape(E_EDGES // SEB, SEB)
    dst2s = edge_index[1].reshape(E_EDGES // SEB, SEB)
    deg16 = _deg_sc(dst2d)
    all_emb, y0a, y0b = _dense0(features, W_red, b_red, conv_weight_0, conv_bias_0)
    s0a, s0b = _spmm_sc(y0a, y0b, src2s, dst2s)
    x1, y1a, y1b = _dense1(s0a, s0b, deg16, conv_weight_1, conv_bias_1)
    s1a, s1b = _spmm_sc(y1a, y1b, src2s, dst2s)
    return _final(all_emb, x1, s1a, s1b, deg16, W_cls, b_cls)


# speedup vs baseline: 1.6883x; 1.0978x over previous
import functools

import jax
import jax.numpy as jnp
from jax import lax
from jax.experimental import pallas as pl
from jax.experimental.pallas import tpu as pltpu
from jax.experimental.pallas import tpu_sc as plsc

N = 50000
N_OUT = 40000
E_EDGES = 800000
D_IN = 128
H_DIM = 64
HH = 32
C_CLS = 50

ROW_BLK = 2000
N_SUBC = 16
NPAD = 50048
TILE_ROWS = NPAD // N_SUBC
ZCHUNK = 184
EB = 80
NBATCH = E_EDGES // EB // N_SUBC
G = 25
NG = NBATCH // G
NBUF = 5
SEB = 80
SNBATCH = E_EDGES // SEB // N_SUBC
SCH = 25
SNCH = SNBATCH // SCH

_sc_mesh = plsc.VectorSubcoreMesh(core_axis_name="c", subcore_axis_name="s")


@functools.partial(
    pl.kernel,
    mesh=_sc_mesh,
    compiler_params=pltpu.CompilerParams(use_tc_tiling_on_sc=False),
    out_type=jax.ShapeDtypeStruct((NPAD, 16), jnp.float32),
    scratch_types=[
        pltpu.VMEM((NBATCH, EB), jnp.int32),
        pltpu.VMEM((EB, 16), jnp.float32),
        pltpu.VMEM((ZCHUNK, 16), jnp.float32),
        pltpu.VMEM_SHARED((NPAD, 16), jnp.float32),
        pltpu.SemaphoreType.DMA,
        pltpu.SemaphoreType.DMA,
        pltpu.SemaphoreType.DMA,
        pltpu.SemaphoreType.DMA,
        pltpu.SemaphoreType.DMA,
    ],
)
def _deg_sc(dst2, out_deg, dbuf, ones_v, zbuf, acc, sem0, sem1, sem2, sem3, sem4):
    c = lax.axis_index("c")
    s = lax.axis_index("s")
    sems = (sem0, sem1, sem2, sem3, sem4)

    @pl.when(c == 0)
    def _():
        def fill_ones(i, carry):
            ones_v[i, :] = jnp.ones((16,), jnp.float32)
            return carry
        lax.fori_loop(0, EB, fill_ones, 0)

        def fill_zero(i, carry):
            zbuf[i, :] = jnp.zeros((16,), jnp.float32)
            return carry
        lax.fori_loop(0, ZCHUNK, fill_zero, 0)

        row0 = s * TILE_ROWS
        def zero_acc(j, carry):
            pltpu.sync_copy(zbuf, acc.at[pl.ds(row0 + j * ZCHUNK, ZCHUNK)])
            return carry
        lax.fori_loop(0, TILE_ROWS // ZCHUNK, zero_acc, 0)
        pltpu.sync_copy(dst2.at[pl.ds(s * NBATCH, NBATCH)], dbuf)
        plsc.subcore_barrier()

        def ws_recon(b):
            pltpu.make_async_copy(ones_v, acc.at[dbuf.at[0]], sems[b]).wait()

        def group(gi, carry):
            base = gi * G
            sc = {}
            for j in range(G):
                b = j % NBUF
                if j < NBUF:
                    pl.when(gi > 0)(functools.partial(ws_recon, b))
                else:
                    sc[j - NBUF].wait()
                sc[j] = pltpu.async_copy(ones_v, acc.at[dbuf.at[base + j]],
                                         sems[b], add=True)
            return carry
        lax.fori_loop(0, NG, group, 0)
        for b in range(NBUF):
            ws_recon(b)
        plsc.subcore_barrier()

        pltpu.sync_copy(acc.at[pl.ds(row0, TILE_ROWS)],
                        out_deg.at[pl.ds(row0, TILE_ROWS)])


@functools.partial(
    pl.kernel,
    mesh=_sc_mesh,
    compiler_params=pltpu.CompilerParams(use_tc_tiling_on_sc=False),
    out_type=[jax.ShapeDtypeStruct((NPAD, HH), jnp.float32),
              jax.ShapeDtypeStruct((NPAD, HH), jnp.float32)],
    scratch_types=[
        pltpu.VMEM((SCH, SEB), jnp.int32),
        pltpu.VMEM((SCH, SEB), jnp.int32),
        pltpu.VMEM((SEB, HH), jnp.float32),
        pltpu.VMEM((SEB, HH), jnp.float32),
        pltpu.VMEM((SEB, HH), jnp.float32),
        pltpu.VMEM((SEB, HH), jnp.float32),
        pltpu.VMEM((ZCHUNK, HH), jnp.float32),
        pltpu.VMEM_SHARED((NPAD, HH), jnp.float32),
        pltpu.SemaphoreType.DMA,
        pltpu.SemaphoreType.DMA,
        pltpu.SemaphoreType.DMA,
        pltpu.SemaphoreType.DMA,
        pltpu.SemaphoreType.DMA,
        pltpu.SemaphoreType.DMA,
        pltpu.SemaphoreType.DMA,
        pltpu.SemaphoreType.DMA,
    ],
)
def _spmm_sc(ya, yb, src2, dst2, out_a, out_b, sbuf, dbuf,
             rows0, rows1, rows2, rows3, zbuf, acc,
             gsem0, gsem1, gsem2, gsem3, ssem0, ssem1, ssem2, ssem3):
    c = lax.axis_index("c")
    s = lax.axis_index("s")
    rows = (rows0, rows1, rows2, rows3)
    gsems = (gsem0, gsem1, gsem2, gsem3)
    ssems = (ssem0, ssem1, ssem2, ssem3)

    def fill_zero(i, carry):
        zbuf[i, pl.ds(0, 16)] = jnp.zeros((16,), jnp.float32)
        zbuf[i, pl.ds(16, 16)] = jnp.zeros((16,), jnp.float32)
        return carry
    lax.fori_loop(0, ZCHUNK, fill_zero, 0)

    row0 = s * TILE_ROWS
    def zero_acc(j, carry):
        pltpu.sync_copy(zbuf, acc.at[pl.ds(row0 + j * ZCHUNK, ZCHUNK)])
        return carry
    lax.fori_loop(0, TILE_ROWS // ZCHUNK, zero_acc, 0)
    plsc.subcore_barrier()

    def edge_pass(y_hbm):
        tile0 = s * SNBATCH
        def chunk(ci, carry):
            pltpu.sync_copy(src2.at[pl.ds(tile0 + ci * SCH, SCH)], sbuf)
            pltpu.sync_copy(dst2.at[pl.ds(tile0 + ci * SCH, SCH)], dbuf)
            g, sc = {}, {}
            def isc(t):
                return pltpu.async_copy(rows[t % 4], acc.at[dbuf.at[t]],
                                        ssems[t % 4], add=True)
            for j in range(SCH):
                b = j % 4
                if j >= 4:
                    sc[j - 4].wait()
                g[j] = pltpu.async_copy(y_hbm.at[sbuf.at[j]], rows[b], gsems[b])
                if j >= 3:
                    g[j - 3].wait()
                    sc[j - 3] = isc(j - 3)
            for t in (SCH - 3, SCH - 2, SCH - 1):
                g[t].wait()
                sc[t] = isc(t)
            sc[SCH - 4].wait()
            sc[SCH - 3].wait()
            sc[SCH - 2].wait()
            sc[SCH - 1].wait()
            return carry
        lax.fori_loop(0, SNCH, chunk, 0)

    pl.when(c == 0)(lambda: edge_pass(ya))
    pl.when(c == 1)(lambda: edge_pass(yb))
    plsc.subcore_barrier()

    pl.when(c == 0)(lambda: pltpu.sync_copy(acc.at[pl.ds(row0, TILE_ROWS)],
                                            out_a.at[pl.ds(row0, TILE_ROWS)]))
    pl.when(c == 1)(lambda: pltpu.sync_copy(acc.at[pl.ds(row0, TILE_ROWS)],
                                            out_b.at[pl.ds(row0, TILE_ROWS)]))


def _dense0_body(f_ref, wred_ref, bred_ref, w0_ref, b0_ref, ae_ref, ya_ref, yb_ref):
    ae = jnp.dot(f_ref[...], wred_ref[...],
                 preferred_element_type=jnp.float32) + bred_ref[...]
    y0 = jnp.dot(ae, w0_ref[...], preferred_element_type=jnp.float32) + b0_ref[...]
    ae_ref[...] = ae
    ya_ref[...] = y0[:, :HH]
    yb_ref[...] = y0[:, HH:]


def _dense0(features, W_red, b_red, W0, b0):
    return pl.pallas_call(
        _dense0_body,
        grid=(N // ROW_BLK,),
        in_specs=[
            pl.BlockSpec((ROW_BLK, D_IN), lambda i: (i, 0)),
            pl.BlockSpec((D_IN, H_DIM), lambda i: (0, 0)),
            pl.BlockSpec((1, H_DIM), lambda i: (0, 0)),
            pl.BlockSpec((H_DIM, H_DIM), lambda i: (0, 0)),
            pl.BlockSpec((1, H_DIM), lambda i: (0, 0)),
        ],
        out_specs=[
            pl.BlockSpec((ROW_BLK, H_DIM), lambda i: (i, 0)),
            pl.BlockSpec((ROW_BLK, HH), lambda i: (i, 0)),
            pl.BlockSpec((ROW_BLK, HH), lambda i: (i, 0)),
        ],
        out_shape=[
            jax.ShapeDtypeStruct((N, H_DIM), jnp.float32),
            jax.ShapeDtypeStruct((N, HH), jnp.float32),
            jax.ShapeDtypeStruct((N, HH), jnp.float32),
        ],
    )(features, W_red, b_red.reshape(1, -1), W0, b0.reshape(1, -1))


def _dense1_body(sa_ref, sb_ref, deg_ref, w1_ref, b1_ref, x1_ref, ya_ref, yb_ref):
    inv = 1.0 / jnp.maximum(deg_ref[:, 0:1], 1.0)
    x1 = jnp.concatenate([sa_ref[...] * inv, sb_ref[...] * inv], axis=1)
    y1 = jnp.dot(x1, w1_ref[...], preferred_element_type=jnp.float32) + b1_ref[...]
    x1_ref[...] = x1
    ya_ref[...] = y1[:, :HH]
    yb_ref[...] = y1[:, HH:]


def _dense1(s0a, s0b, deg16, W1, b1):
    return pl.pallas_call(
        _dense1_body,
        grid=(N // ROW_BLK,),
        in_specs=[
            pl.BlockSpec((ROW_BLK, HH), lambda i: (i, 0)),
            pl.BlockSpec((ROW_BLK, HH), lambda i: (i, 0)),
            pl.BlockSpec((ROW_BLK, 16), lambda i: (i, 0)),
            pl.BlockSpec((H_DIM, H_DIM), lambda i: (0, 0)),
            pl.BlockSpec((1, H_DIM), lambda i: (0, 0)),
        ],
        out_specs=[
            pl.BlockSpec((ROW_BLK, H_DIM), lambda i: (i, 0)),
            pl.BlockSpec((ROW_BLK, HH), lambda i: (i, 0)),
            pl.BlockSpec((ROW_BLK, HH), lambda i: (i, 0)),
        ],
        out_shape=[
            jax.ShapeDtypeStruct((N, H_DIM), jnp.float32),
            jax.ShapeDtypeStruct((N, HH), jnp.float32),
            jax.ShapeDtypeStruct((N, HH), jnp.float32),
        ],
    )(s0a, s0b, deg16, W1, b1.reshape(1, -1))


def _final_body(ae_ref, x1_ref, sa_ref, sb_ref, deg_ref, wc_ref, bc_ref, out_ref):
    inv = 1.0 / jnp.maximum(deg_ref[:, 0:1], 1.0)
    x2 = jnp.concatenate([sa_ref[...] * inv, sb_ref[...] * inv], axis=1)
    m = (ae_ref[...] + x1_ref[...] + x2) * (1.0 / 3.0)
    z = jnp.dot(m, wc_ref[...], preferred_element_type=jnp.float32) + bc_ref[...]
    zmax = jnp.max(z, axis=1, keepdims=True)
    lse = jnp.log(jnp.sum(jnp.exp(z - zmax), axis=1, keepdims=True)) + zmax
    out_ref[...] = z - lse


def _final(all_emb, x1, s1a, s1b, deg16, W_cls, b_cls):
    return pl.pallas_call(
        _final_body,
        grid=(N_OUT // ROW_BLK,),
        in_specs=[
            pl.BlockSpec((ROW_BLK, H_DIM), lambda i: (i, 0)),
            pl.BlockSpec((ROW_BLK, H_DIM), lambda i: (i, 0)),
            pl.BlockSpec((ROW_BLK, HH), lambda i: (i, 0)),
            pl.BlockSpec((ROW_BLK, HH), lambda i: (i, 0)),
            pl.BlockSpec((ROW_BLK, 16), lambda i: (i, 0)),
            pl.BlockSpec((H_DIM, C_CLS), lambda i: (0, 0)),
            pl.BlockSpec((1, C_CLS), lambda i: (0, 0)),
        ],
        out_specs=pl.BlockSpec((ROW_BLK, C_CLS), lambda i: (i, 0)),
        out_shape=jax.ShapeDtypeStruct((N_OUT, C_CLS), jnp.float32),
    )(all_emb, x1, s1a, s1b, deg16, W_cls, b_cls.reshape(1, -1))


def kernel(features, edge_index, W_red, b_red, conv_weight_0, conv_bias_0,
           conv_weight_1, conv_bias_1, W_cls, b_cls):
    dst2d = edge_index[1].reshape(E_EDGES // EB, EB)
    src2s = edge_index[0].reshape(E_EDGES // SEB, SEB)
    dst2s = edge_index[1].reshape(E_EDGES // SEB, SEB)
    deg16 = _deg_sc(dst2d)
    all_emb, y0a, y0b = _dense0(features, W_red, b_red, conv_weight_0, conv_bias_0)
    s0a, s0b = _spmm_sc(y0a, y0b, src2s, dst2s)
    x1, y1a, y1b = _dense1(s0a, s0b, deg16, conv_weight_1, conv_bias_1)
    s1a, s1b = _spmm_sc(y1a, y1b, src2s, dst2s)
    return _final(all_emb, x1, s1a, s1b, deg16, W_cls, b_cls)


# speedup vs baseline: 1.8303x; 1.0841x over previous
import functools

import jax
import jax.numpy as jnp
from jax import lax
from jax.experimental import pallas as pl
from jax.experimental.pallas import tpu as pltpu
from jax.experimental.pallas import tpu_sc as plsc

N = 50000
N_OUT = 40000
E_EDGES = 800000
D_IN = 128
H_DIM = 64
HH = 32
C_CLS = 50

ROW_BLK = 2000
N_SUBC = 16
NPAD = 50048
TILE_ROWS = NPAD // N_SUBC
ZCHUNK = 184
EB = 80
NBATCH = E_EDGES // EB // N_SUBC
G = 25
NG = NBATCH // G
NBUF = 5
SEB = 80
SNBATCH = E_EDGES // SEB // N_SUBC
SCH = 25
SNCH = SNBATCH // SCH
RING = 6
LAG = RING - 1

_sc_mesh = plsc.VectorSubcoreMesh(core_axis_name="c", subcore_axis_name="s")


@functools.partial(
    pl.kernel,
    mesh=_sc_mesh,
    compiler_params=pltpu.CompilerParams(use_tc_tiling_on_sc=False),
    out_type=jax.ShapeDtypeStruct((NPAD, 16), jnp.float32),
    scratch_types=[
        pltpu.VMEM((NBATCH, EB), jnp.int32),
        pltpu.VMEM((EB, 16), jnp.float32),
        pltpu.VMEM((ZCHUNK, 16), jnp.float32),
        pltpu.VMEM_SHARED((NPAD, 16), jnp.float32),
        pltpu.SemaphoreType.DMA,
        pltpu.SemaphoreType.DMA,
        pltpu.SemaphoreType.DMA,
        pltpu.SemaphoreType.DMA,
        pltpu.SemaphoreType.DMA,
    ],
)
def _deg_sc(dst2, out_deg, dbuf, ones_v, zbuf, acc, sem0, sem1, sem2, sem3, sem4):
    c = lax.axis_index("c")
    s = lax.axis_index("s")
    sems = (sem0, sem1, sem2, sem3, sem4)

    @pl.when(c == 0)
    def _():
        def fill_ones(i, carry):
            ones_v[i, :] = jnp.ones((16,), jnp.float32)
            return carry
        lax.fori_loop(0, EB, fill_ones, 0)

        def fill_zero(i, carry):
            zbuf[i, :] = jnp.zeros((16,), jnp.float32)
            return carry
        lax.fori_loop(0, ZCHUNK, fill_zero, 0)

        row0 = s * TILE_ROWS
        def zero_acc(j, carry):
            pltpu.sync_copy(zbuf, acc.at[pl.ds(row0 + j * ZCHUNK, ZCHUNK)])
            return carry
        lax.fori_loop(0, TILE_ROWS // ZCHUNK, zero_acc, 0)
        pltpu.sync_copy(dst2.at[pl.ds(s * NBATCH, NBATCH)], dbuf)
        plsc.subcore_barrier()

        def ws_recon(b):
            pltpu.make_async_copy(ones_v, acc.at[dbuf.at[0]], sems[b]).wait()

        def group(gi, carry):
            base = gi * G
            sc = {}
            for j in range(G):
                b = j % NBUF
                if j < NBUF:
                    pl.when(gi > 0)(functools.partial(ws_recon, b))
                else:
                    sc[j - NBUF].wait()
                sc[j] = pltpu.async_copy(ones_v, acc.at[dbuf.at[base + j]],
                                         sems[b], add=True)
            return carry
        lax.fori_loop(0, NG, group, 0)
        for b in range(NBUF):
            ws_recon(b)
        plsc.subcore_barrier()

        pltpu.sync_copy(acc.at[pl.ds(row0, TILE_ROWS)],
                        out_deg.at[pl.ds(row0, TILE_ROWS)])


@functools.partial(
    pl.kernel,
    mesh=_sc_mesh,
    compiler_params=pltpu.CompilerParams(use_tc_tiling_on_sc=False),
    out_type=[jax.ShapeDtypeStruct((NPAD, HH), jnp.float32),
              jax.ShapeDtypeStruct((NPAD, HH), jnp.float32)],
    scratch_types=[
        pltpu.VMEM((SCH, SEB), jnp.int32),
        pltpu.VMEM((SCH, SEB), jnp.int32),
        *([pltpu.VMEM((SEB, HH), jnp.float32)] * RING),
        pltpu.VMEM((ZCHUNK, HH), jnp.float32),
        pltpu.VMEM_SHARED((NPAD, HH), jnp.float32),
        *([pltpu.SemaphoreType.DMA] * (2 * RING)),
    ],
)
def _spmm_sc(ya, yb, src2, dst2, out_a, out_b, sbuf, dbuf, *rest):
    rows = rest[:RING]
    zbuf = rest[RING]
    acc = rest[RING + 1]
    gsems = rest[RING + 2:2 * RING + 2]
    ssems = rest[2 * RING + 2:]
    c = lax.axis_index("c")
    s = lax.axis_index("s")

    def fill_zero(i, carry):
        zbuf[i, pl.ds(0, 16)] = jnp.zeros((16,), jnp.float32)
        zbuf[i, pl.ds(16, 16)] = jnp.zeros((16,), jnp.float32)
        return carry
    lax.fori_loop(0, ZCHUNK, fill_zero, 0)

    row0 = s * TILE_ROWS
    def zero_acc(j, carry):
        pltpu.sync_copy(zbuf, acc.at[pl.ds(row0 + j * ZCHUNK, ZCHUNK)])
        return carry
    lax.fori_loop(0, TILE_ROWS // ZCHUNK, zero_acc, 0)
    plsc.subcore_barrier()

    def edge_pass(y_hbm):
        tile0 = s * SNBATCH
        def chunk(ci, carry):
            pltpu.sync_copy(src2.at[pl.ds(tile0 + ci * SCH, SCH)], sbuf)
            pltpu.sync_copy(dst2.at[pl.ds(tile0 + ci * SCH, SCH)], dbuf)
            g, sc = {}, {}
            def isc(t):
                return pltpu.async_copy(rows[t % RING], acc.at[dbuf.at[t]],
                                        ssems[t % RING], add=True)
            for j in range(SCH):
                b = j % RING
                if j >= RING:
                    sc[j - RING].wait()
                g[j] = pltpu.async_copy(y_hbm.at[sbuf.at[j]], rows[b], gsems[b])
                if j >= LAG:
                    g[j - LAG].wait()
                    sc[j - LAG] = isc(j - LAG)
            for t in range(SCH - LAG, SCH):
                g[t].wait()
                sc[t] = isc(t)
            for t in range(SCH - RING, SCH):
                sc[t].wait()
            return carry
        lax.fori_loop(0, SNCH, chunk, 0)

    pl.when(c == 0)(lambda: edge_pass(ya))
    pl.when(c == 1)(lambda: edge_pass(yb))
    plsc.subcore_barrier()

    pl.when(c == 0)(lambda: pltpu.sync_copy(acc.at[pl.ds(row0, TILE_ROWS)],
                                            out_a.at[pl.ds(row0, TILE_ROWS)]))
    pl.when(c == 1)(lambda: pltpu.sync_copy(acc.at[pl.ds(row0, TILE_ROWS)],
                                            out_b.at[pl.ds(row0, TILE_ROWS)]))


def _dense0_body(f_ref, wred_ref, bred_ref, w0_ref, b0_ref, ae_ref, ya_ref, yb_ref):
    ae = jnp.dot(f_ref[...], wred_ref[...],
                 preferred_element_type=jnp.float32) + bred_ref[...]
    y0 = jnp.dot(ae, w0_ref[...], preferred_element_type=jnp.float32) + b0_ref[...]
    ae_ref[...] = ae
    ya_ref[...] = y0[:, :HH]
    yb_ref[...] = y0[:, HH:]


def _dense0(features, W_red, b_red, W0, b0):
    return pl.pallas_call(
        _dense0_body,
        grid=(N // ROW_BLK,),
        in_specs=[
            pl.BlockSpec((ROW_BLK, D_IN), lambda i: (i, 0)),
            pl.BlockSpec((D_IN, H_DIM), lambda i: (0, 0)),
            pl.BlockSpec((1, H_DIM), lambda i: (0, 0)),
            pl.BlockSpec((H_DIM, H_DIM), lambda i: (0, 0)),
            pl.BlockSpec((1, H_DIM), lambda i: (0, 0)),
        ],
        out_specs=[
            pl.BlockSpec((ROW_BLK, H_DIM), lambda i: (i, 0)),
            pl.BlockSpec((ROW_BLK, HH), lambda i: (i, 0)),
            pl.BlockSpec((ROW_BLK, HH), lambda i: (i, 0)),
        ],
        out_shape=[
            jax.ShapeDtypeStruct((N, H_DIM), jnp.float32),
            jax.ShapeDtypeStruct((N, HH), jnp.float32),
            jax.ShapeDtypeStruct((N, HH), jnp.float32),
        ],
    )(features, W_red, b_red.reshape(1, -1), W0, b0.reshape(1, -1))


def _dense1_body(sa_ref, sb_ref, deg_ref, w1_ref, b1_ref, x1_ref, ya_ref, yb_ref):
    inv = 1.0 / jnp.maximum(deg_ref[:, 0:1], 1.0)
    x1 = jnp.concatenate([sa_ref[...] * inv, sb_ref[...] * inv], axis=1)
    y1 = jnp.dot(x1, w1_ref[...], preferred_element_type=jnp.float32) + b1_ref[...]
    x1_ref[...] = x1
    ya_ref[...] = y1[:, :HH]
    yb_ref[...] = y1[:, HH:]


def _dense1(s0a, s0b, deg16, W1, b1):
    return pl.pallas_call(
        _dense1_body,
        grid=(N // ROW_BLK,),
        in_specs=[
            pl.BlockSpec((ROW_BLK, HH), lambda i: (i, 0)),
            pl.BlockSpec((ROW_BLK, HH), lambda i: (i, 0)),
            pl.BlockSpec((ROW_BLK, 16), lambda i: (i, 0)),
            pl.BlockSpec((H_DIM, H_DIM), lambda i: (0, 0)),
            pl.BlockSpec((1, H_DIM), lambda i: (0, 0)),
        ],
        out_specs=[
            pl.BlockSpec((ROW_BLK, H_DIM), lambda i: (i, 0)),
            pl.BlockSpec((ROW_BLK, HH), lambda i: (i, 0)),
            pl.BlockSpec((ROW_BLK, HH), lambda i: (i, 0)),
        ],
        out_shape=[
            jax.ShapeDtypeStruct((N, H_DIM), jnp.float32),
            jax.ShapeDtypeStruct((N, HH), jnp.float32),
            jax.ShapeDtypeStruct((N, HH), jnp.float32),
        ],
    )(s0a, s0b, deg16, W1, b1.reshape(1, -1))


def _final_body(ae_ref, x1_ref, sa_ref, sb_ref, deg_ref, wc_ref, bc_ref, out_ref):
    inv = 1.0 / jnp.maximum(deg_ref[:, 0:1], 1.0)
    x2 = jnp.concatenate([sa_ref[...] * inv, sb_ref[...] * inv], axis=1)
    m = (ae_ref[...] + x1_ref[...] + x2) * (1.0 / 3.0)
    z = jnp.dot(m, wc_ref[...], preferred_element_type=jnp.float32) + bc_ref[...]
    zmax = jnp.max(z, axis=1, keepdims=True)
    lse = jnp.log(jnp.sum(jnp.exp(z - zmax), axis=1, keepdims=True)) + zmax
    out_ref[...] = z - lse


def _final(all_emb, x1, s1a, s1b, deg16, W_cls, b_cls):
    return pl.pallas_call(
        _final_body,
        grid=(N_OUT // ROW_BLK,),
        in_specs=[
            pl.BlockSpec((ROW_BLK, H_DIM), lambda i: (i, 0)),
            pl.BlockSpec((ROW_BLK, H_DIM), lambda i: (i, 0)),
            pl.BlockSpec((ROW_BLK, HH), lambda i: (i, 0)),
            pl.BlockSpec((ROW_BLK, HH), lambda i: (i, 0)),
            pl.BlockSpec((ROW_BLK, 16), lambda i: (i, 0)),
            pl.BlockSpec((H_DIM, C_CLS), lambda i: (0, 0)),
            pl.BlockSpec((1, C_CLS), lambda i: (0, 0)),
        ],
        out_specs=pl.BlockSpec((ROW_BLK, C_CLS), lambda i: (i, 0)),
        out_shape=jax.ShapeDtypeStruct((N_OUT, C_CLS), jnp.float32),
    )(all_emb, x1, s1a, s1b, deg16, W_cls, b_cls.reshape(1, -1))


def kernel(features, edge_index, W_red, b_red, conv_weight_0, conv_bias_0,
           conv_weight_1, conv_bias_1, W_cls, b_cls):
    dst2d = edge_index[1].reshape(E_EDGES // EB, EB)
    src2s = edge_index[0].reshape(E_EDGES // SEB, SEB)
    dst2s = edge_index[1].reshape(E_EDGES // SEB, SEB)
    deg16 = _deg_sc(dst2d)
    all_emb, y0a, y0b = _dense0(features, W_red, b_red, conv_weight_0, conv_bias_0)
    s0a, s0b = _spmm_sc(y0a, y0b, src2s, dst2s)
    x1, y1a, y1b = _dense1(s0a, s0b, deg16, conv_weight_1, conv_bias_1)
    s1a, s1b = _spmm_sc(y1a, y1b, src2s, dst2s)
    return _final(all_emb, x1, s1a, s1b, deg16, W_cls, b_cls)


# speedup vs baseline: 1.8421x; 1.0065x over previous
import functools

import jax
import jax.numpy as jnp
from jax import lax
from jax.experimental import pallas as pl
from jax.experimental.pallas import tpu as pltpu
from jax.experimental.pallas import tpu_sc as plsc

N = 50000
N_OUT = 40000
E_EDGES = 800000
D_IN = 128
H_DIM = 64
HH = 32
C_CLS = 50

ROW_BLK = 2000
N_SUBC = 16
NPAD = 50048
TILE_ROWS = NPAD // N_SUBC
ZCHUNK = 184
EB = 80
NBATCH = E_EDGES // EB // N_SUBC
G = 25
NG = NBATCH // G
NBUF = 5
SEB = 80
SNBATCH = E_EDGES // SEB // N_SUBC
SCH = 25
SNCH = SNBATCH // SCH
RING = 8
LAG = RING - 1

_sc_mesh = plsc.VectorSubcoreMesh(core_axis_name="c", subcore_axis_name="s")


@functools.partial(
    pl.kernel,
    mesh=_sc_mesh,
    compiler_params=pltpu.CompilerParams(use_tc_tiling_on_sc=False),
    out_type=jax.ShapeDtypeStruct((NPAD, 16), jnp.float32),
    scratch_types=[
        pltpu.VMEM((NBATCH, EB), jnp.int32),
        pltpu.VMEM((EB, 16), jnp.float32),
        pltpu.VMEM((ZCHUNK, 16), jnp.float32),
        pltpu.VMEM_SHARED((NPAD, 16), jnp.float32),
        pltpu.SemaphoreType.DMA,
        pltpu.SemaphoreType.DMA,
        pltpu.SemaphoreType.DMA,
        pltpu.SemaphoreType.DMA,
        pltpu.SemaphoreType.DMA,
    ],
)
def _deg_sc(dst2, out_deg, dbuf, ones_v, zbuf, acc, sem0, sem1, sem2, sem3, sem4):
    c = lax.axis_index("c")
    s = lax.axis_index("s")
    sems = (sem0, sem1, sem2, sem3, sem4)

    @pl.when(c == 0)
    def _():
        def fill_ones(i, carry):
            ones_v[i, :] = jnp.ones((16,), jnp.float32)
            return carry
        lax.fori_loop(0, EB, fill_ones, 0)

        def fill_zero(i, carry):
            zbuf[i, :] = jnp.zeros((16,), jnp.float32)
            return carry
        lax.fori_loop(0, ZCHUNK, fill_zero, 0)

        row0 = s * TILE_ROWS
        def zero_acc(j, carry):
            pltpu.sync_copy(zbuf, acc.at[pl.ds(row0 + j * ZCHUNK, ZCHUNK)])
            return carry
        lax.fori_loop(0, TILE_ROWS // ZCHUNK, zero_acc, 0)
        pltpu.sync_copy(dst2.at[pl.ds(s * NBATCH, NBATCH)], dbuf)
        plsc.subcore_barrier()

        def ws_recon(b):
            pltpu.make_async_copy(ones_v, acc.at[dbuf.at[0]], sems[b]).wait()

        def group(gi, carry):
            base = gi * G
            sc = {}
            for j in range(G):
                b = j % NBUF
                if j < NBUF:
                    pl.when(gi > 0)(functools.partial(ws_recon, b))
                else:
                    sc[j - NBUF].wait()
                sc[j] = pltpu.async_copy(ones_v, acc.at[dbuf.at[base + j]],
                                         sems[b], add=True)
            return carry
        lax.fori_loop(0, NG, group, 0)
        for b in range(NBUF):
            ws_recon(b)
        plsc.subcore_barrier()

        pltpu.sync_copy(acc.at[pl.ds(row0, TILE_ROWS)],
                        out_deg.at[pl.ds(row0, TILE_ROWS)])


@functools.partial(
    pl.kernel,
    mesh=_sc_mesh,
    compiler_params=pltpu.CompilerParams(use_tc_tiling_on_sc=False),
    out_type=[jax.ShapeDtypeStruct((NPAD, HH), jnp.float32),
              jax.ShapeDtypeStruct((NPAD, HH), jnp.float32)],
    scratch_types=[
        pltpu.VMEM((SCH, SEB), jnp.int32),
        pltpu.VMEM((SCH, SEB), jnp.int32),
        *([pltpu.VMEM((SEB, HH), jnp.float32)] * RING),
        pltpu.VMEM((ZCHUNK, HH), jnp.float32),
        pltpu.VMEM_SHARED((NPAD, HH), jnp.float32),
        *([pltpu.SemaphoreType.DMA] * (2 * RING)),
    ],
)
def _spmm_sc(ya, yb, src2, dst2, out_a, out_b, sbuf, dbuf, *rest):
    rows = rest[:RING]
    zbuf = rest[RING]
    acc = rest[RING + 1]
    gsems = rest[RING + 2:2 * RING + 2]
    ssems = rest[2 * RING + 2:]
    c = lax.axis_index("c")
    s = lax.axis_index("s")

    def fill_zero(i, carry):
        zbuf[i, pl.ds(0, 16)] = jnp.zeros((16,), jnp.float32)
        zbuf[i, pl.ds(16, 16)] = jnp.zeros((16,), jnp.float32)
        return carry
    lax.fori_loop(0, ZCHUNK, fill_zero, 0)

    row0 = s * TILE_ROWS
    def zero_acc(j, carry):
        pltpu.sync_copy(zbuf, acc.at[pl.ds(row0 + j * ZCHUNK, ZCHUNK)])
        return carry
    lax.fori_loop(0, TILE_ROWS // ZCHUNK, zero_acc, 0)
    plsc.subcore_barrier()

    def edge_pass(y_hbm):
        tile0 = s * SNBATCH
        def chunk(ci, carry):
            pltpu.sync_copy(src2.at[pl.ds(tile0 + ci * SCH, SCH)], sbuf)
            pltpu.sync_copy(dst2.at[pl.ds(tile0 + ci * SCH, SCH)], dbuf)
            g, sc = {}, {}
            def isc(t):
                return pltpu.async_copy(rows[t % RING], acc.at[dbuf.at[t]],
                                        ssems[t % RING], add=True)
            for j in range(SCH):
                b = j % RING
                if j >= RING:
                    sc[j - RING].wait()
                g[j] = pltpu.async_copy(y_hbm.at[sbuf.at[j]], rows[b], gsems[b])
                if j >= LAG:
                    g[j - LAG].wait()
                    sc[j - LAG] = isc(j - LAG)
            for t in range(SCH - LAG, SCH):
                g[t].wait()
                sc[t] = isc(t)
            for t in range(SCH - RING, SCH):
                sc[t].wait()
            return carry
        lax.fori_loop(0, SNCH, chunk, 0)

    pl.when(c == 0)(lambda: edge_pass(ya))
    pl.when(c == 1)(lambda: edge_pass(yb))
    plsc.subcore_barrier()

    pl.when(c == 0)(lambda: pltpu.sync_copy(acc.at[pl.ds(row0, TILE_ROWS)],
                                            out_a.at[pl.ds(row0, TILE_ROWS)]))
    pl.when(c == 1)(lambda: pltpu.sync_copy(acc.at[pl.ds(row0, TILE_ROWS)],
                                            out_b.at[pl.ds(row0, TILE_ROWS)]))


def _dense0_body(f_ref, wred_ref, bred_ref, w0_ref, b0_ref, ae_ref, ya_ref, yb_ref):
    ae = jnp.dot(f_ref[...], wred_ref[...],
                 preferred_element_type=jnp.float32) + bred_ref[...]
    y0 = jnp.dot(ae, w0_ref[...], preferred_element_type=jnp.float32) + b0_ref[...]
    ae_ref[...] = ae
    ya_ref[...] = y0[:, :HH]
    yb_ref[...] = y0[:, HH:]


def _dense0(features, W_red, b_red, W0, b0):
    return pl.pallas_call(
        _dense0_body,
        grid=(N // ROW_BLK,),
        in_specs=[
            pl.BlockSpec((ROW_BLK, D_IN), lambda i: (i, 0)),
            pl.BlockSpec((D_IN, H_DIM), lambda i: (0, 0)),
            pl.BlockSpec((1, H_DIM), lambda i: (0, 0)),
            pl.BlockSpec((H_DIM, H_DIM), lambda i: (0, 0)),
            pl.BlockSpec((1, H_DIM), lambda i: (0, 0)),
        ],
        out_specs=[
            pl.BlockSpec((ROW_BLK, H_DIM), lambda i: (i, 0)),
            pl.BlockSpec((ROW_BLK, HH), lambda i: (i, 0)),
            pl.BlockSpec((ROW_BLK, HH), lambda i: (i, 0)),
        ],
        out_shape=[
            jax.ShapeDtypeStruct((N, H_DIM), jnp.float32),
            jax.ShapeDtypeStruct((N, HH), jnp.float32),
            jax.ShapeDtypeStruct((N, HH), jnp.float32),
        ],
    )(features, W_red, b_red.reshape(1, -1), W0, b0.reshape(1, -1))


def _dense1_body(sa_ref, sb_ref, deg_ref, w1_ref, b1_ref, x1_ref, ya_ref, yb_ref):
    inv = 1.0 / jnp.maximum(deg_ref[:, 0:1], 1.0)
    x1 = jnp.concatenate([sa_ref[...] * inv, sb_ref[...] * inv], axis=1)
    y1 = jnp.dot(x1, w1_ref[...], preferred_element_type=jnp.float32) + b1_ref[...]
    x1_ref[...] = x1
    ya_ref[...] = y1[:, :HH]
    yb_ref[...] = y1[:, HH:]


def _dense1(s0a, s0b, deg16, W1, b1):
    return pl.pallas_call(
        _dense1_body,
        grid=(N // ROW_BLK,),
        in_specs=[
            pl.BlockSpec((ROW_BLK, HH), lambda i: (i, 0)),
            pl.BlockSpec((ROW_BLK, HH), lambda i: (i, 0)),
            pl.BlockSpec((ROW_BLK, 16), lambda i: (i, 0)),
            pl.BlockSpec((H_DIM, H_DIM), lambda i: (0, 0)),
            pl.BlockSpec((1, H_DIM), lambda i: (0, 0)),
        ],
        out_specs=[
            pl.BlockSpec((ROW_BLK, H_DIM), lambda i: (i, 0)),
            pl.BlockSpec((ROW_BLK, HH), lambda i: (i, 0)),
            pl.BlockSpec((ROW_BLK, HH), lambda i: (i, 0)),
        ],
        out_shape=[
            jax.ShapeDtypeStruct((N, H_DIM), jnp.float32),
            jax.ShapeDtypeStruct((N, HH), jnp.float32),
            jax.ShapeDtypeStruct((N, HH), jnp.float32),
        ],
    )(s0a, s0b, deg16, W1, b1.reshape(1, -1))


def _final_body(ae_ref, x1_ref, sa_ref, sb_ref, deg_ref, wc_ref, bc_ref, out_ref):
    inv = 1.0 / jnp.maximum(deg_ref[:, 0:1], 1.0)
    x2 = jnp.concatenate([sa_ref[...] * inv, sb_ref[...] * inv], axis=1)
    m = (ae_ref[...] + x1_ref[...] + x2) * (1.0 / 3.0)
    z = jnp.dot(m, wc_ref[...], preferred_element_type=jnp.float32) + bc_ref[...]
    zmax = jnp.max(z, axis=1, keepdims=True)
    lse = jnp.log(jnp.sum(jnp.exp(z - zmax), axis=1, keepdims=True)) + zmax
    out_ref[...] = z - lse


def _final(all_emb, x1, s1a, s1b, deg16, W_cls, b_cls):
    return pl.pallas_call(
        _final_body,
        grid=(N_OUT // ROW_BLK,),
        in_specs=[
            pl.BlockSpec((ROW_BLK, H_DIM), lambda i: (i, 0)),
            pl.BlockSpec((ROW_BLK, H_DIM), lambda i: (i, 0)),
            pl.BlockSpec((ROW_BLK, HH), lambda i: (i, 0)),
            pl.BlockSpec((ROW_BLK, HH), lambda i: (i, 0)),
            pl.BlockSpec((ROW_BLK, 16), lambda i: (i, 0)),
            pl.BlockSpec((H_DIM, C_CLS), lambda i: (0, 0)),
            pl.BlockSpec((1, C_CLS), lambda i: (0, 0)),
        ],
        out_specs=pl.BlockSpec((ROW_BLK, C_CLS), lambda i: (i, 0)),
        out_shape=jax.ShapeDtypeStruct((N_OUT, C_CLS), jnp.float32),
    )(all_emb, x1, s1a, s1b, deg16, W_cls, b_cls.reshape(1, -1))


def kernel(features, edge_index, W_red, b_red, conv_weight_0, conv_bias_0,
           conv_weight_1, conv_bias_1, W_cls, b_cls):
    dst2d = edge_index[1].reshape(E_EDGES // EB, EB)
    src2s = edge_index[0].reshape(E_EDGES // SEB, SEB)
    dst2s = edge_index[1].reshape(E_EDGES // SEB, SEB)
    deg16 = _deg_sc(dst2d)
    all_emb, y0a, y0b = _dense0(features, W_red, b_red, conv_weight_0, conv_bias_0)
    s0a, s0b = _spmm_sc(y0a, y0b, src2s, dst2s)
    x1, y1a, y1b = _dense1(s0a, s0b, deg16, conv_weight_1, conv_bias_1)
    s1a, s1b = _spmm_sc(y1a, y1b, src2s, dst2s)
    return _final(all_emb, x1, s1a, s1b, deg16, W_cls, b_cls)
